# Initial kernel scaffold; baseline (speedup 1.0000x reference)
import functools

import jax
import jax.numpy as jnp
from jax import lax
from jax.experimental import pallas as pl
from jax.experimental.pallas import tpu as pltpu
from jax.experimental.pallas import tpu_sc as plsc

N = 50000
E = 1600000
N_PAD = 50176
NW = 32
CHUNK = 128
GRP = 49
NGRP = 8
E_PER_TILE = CHUNK * GRP * NGRP
E_PAD = NW * E_PER_TILE
ROWS_PER_TILE = N_PAD // 16
TRASH = N
K1 = 25000
K2 = 12500
BN = 1024
NBLK = N_PAD // BN

_mesh = plsc.VectorSubcoreMesh(core_axis_name="c", subcore_axis_name="s")


def _deg_body(srcr, dstr, zeros1, out, degS, degD, ones_v, idx_v):
    c = lax.axis_index("c")
    s = lax.axis_index("s")
    w = c * 16 + s

    def setones(j, _):
        ones_v[pl.ds(j * 16, 16)] = jnp.ones((16,), jnp.float32)
        return 0
    lax.fori_loop(0, CHUNK // 16, setones, 0)

    row0 = s * ROWS_PER_TILE
    pltpu.sync_copy(zeros1.at[pl.ds(row0, ROWS_PER_TILE)],
                    degS.at[pl.ds(row0, ROWS_PER_TILE)])
    pltpu.sync_copy(zeros1.at[pl.ds(row0, ROWS_PER_TILE)],
                    degD.at[pl.ds(row0, ROWS_PER_TILE)])
    plsc.subcore_barrier()

    def grp(g, _):
        pltpu.sync_copy(srcr.at[w, pl.ds(g * GRP, GRP)], idx_v)

        def chunk_s(j, _):
            pltpu.sync_copy(ones_v, degS.at[idx_v.at[j]], add=True)
            return 0
        lax.fori_loop(0, GRP, chunk_s, 0)
        pltpu.sync_copy(dstr.at[w, pl.ds(g * GRP, GRP)], idx_v)

        def chunk_d(j, _):
            pltpu.sync_copy(ones_v, degD.at[idx_v.at[j]], add=True)
            return 0
        lax.fori_loop(0, GRP, chunk_d, 0)
        return 0
    lax.fori_loop(0, NGRP, grp, 0)

    plsc.subcore_barrier()
    pltpu.sync_copy(degS.at[pl.ds(row0, ROWS_PER_TILE)],
                    out.at[c, 0, pl.ds(row0, ROWS_PER_TILE)])
    pltpu.sync_copy(degD.at[pl.ds(row0, ROWS_PER_TILE)],
                    out.at[c, 1, pl.ds(row0, ROWS_PER_TILE)])


_sc_deg = functools.partial(
    pl.kernel, _deg_body,
    out_type=jax.ShapeDtypeStruct((2, 2, N_PAD), jnp.float32),
    mesh=_mesh,
    scratch_types=[
        pltpu.VMEM_SHARED((N_PAD,), jnp.float32),
        pltpu.VMEM_SHARED((N_PAD,), jnp.float32),
        pltpu.VMEM((CHUNK,), jnp.float32),
        pltpu.VMEM((GRP, CHUNK), jnp.int32),
    ],
)()


def _agg_body(D, srcr, dstr, tab, zerosD, out, acc, idxs_v, idxd_v, rows_v,
              sem):
    c = lax.axis_index("c")
    s = lax.axis_index("s")
    w = c * 16 + s

    row0 = s * ROWS_PER_TILE
    pltpu.sync_copy(zerosD.at[pl.ds(row0, ROWS_PER_TILE)],
                    acc.at[pl.ds(row0, ROWS_PER_TILE)])
    plsc.subcore_barrier()

    def grp(g, _):
        pltpu.sync_copy(srcr.at[w, pl.ds(g * GRP, GRP)], idxs_v)
        pltpu.sync_copy(dstr.at[w, pl.ds(g * GRP, GRP)], idxd_v)

        def chunk(j, _):
            pltpu.async_copy(tab.at[idxs_v.at[j]], rows_v, sem).wait()
            pltpu.sync_copy(rows_v, acc.at[idxd_v.at[j]], add=True)
            return 0
        lax.fori_loop(0, GRP, chunk, 0)
        return 0
    lax.fori_loop(0, NGRP, grp, 0)

    plsc.subcore_barrier()
    pltpu.sync_copy(acc.at[pl.ds(row0, ROWS_PER_TILE)],
                    out.at[c, pl.ds(row0, ROWS_PER_TILE)])


def _make_sc_agg(D):
    return functools.partial(
        pl.kernel, functools.partial(_agg_body, D),
        out_type=jax.ShapeDtypeStruct((2, N_PAD, D), jnp.float32),
        mesh=_mesh,
        scratch_types=[
            pltpu.VMEM_SHARED((N_PAD, D), jnp.float32),
            pltpu.VMEM((GRP, CHUNK), jnp.int32),
            pltpu.VMEM((GRP, CHUNK), jnp.int32),
            pltpu.VMEM((CHUNK, D), jnp.float32),
            pltpu.SemaphoreType.DMA,
        ],
    )()


_sc_agg4 = _make_sc_agg(4)
_sc_agg32 = _make_sc_agg(32)


def _u32(y):
    u = lax.bitcast_convert_type(y, jnp.uint32)
    return jnp.where((u >> jnp.uint32(31)) > jnp.uint32(0),
                     ~u, u | jnp.uint32(0x80000000))


def _deg_tc_body(deg_ref, x4_ref, a0_o, b0_o, t0_o):
    ds = deg_ref[0, 0] + deg_ref[1, 0]
    dd = deg_ref[0, 1] + deg_ref[1, 1]
    a0 = jnp.where(ds > 0, lax.rsqrt(jnp.maximum(ds, 1e-12)),
                   jnp.float32(0))
    b0 = jnp.where(dd > 0, lax.rsqrt(jnp.maximum(dd, 1e-12)),
                   jnp.float32(0))
    a0_o[...] = a0
    b0_o[...] = b0
    t0_o[...] = x4_ref[...] * a0


def _tc_deg(deg, x4):
    return pl.pallas_call(
        _deg_tc_body,
        grid=(NBLK,),
        in_specs=[
            pl.BlockSpec((2, 2, BN, 1), lambda i: (0, 0, i, 0)),
            pl.BlockSpec((BN, 4), lambda i: (i, 0)),
        ],
        out_specs=[
            pl.BlockSpec((BN, 1), lambda i: (i, 0)),
            pl.BlockSpec((BN, 1), lambda i: (i, 0)),
            pl.BlockSpec((BN, 4), lambda i: (i, 0)),
        ],
        out_shape=[
            jax.ShapeDtypeStruct((N_PAD, 1), jnp.float32),
            jax.ShapeDtypeStruct((N_PAD, 1), jnp.float32),
            jax.ShapeDtypeStruct((N_PAD, 4), jnp.float32),
        ],
    )(deg.reshape(2, 2, N_PAD, 1), x4)


def _layer_body(aggp_ref, bm_ref, xin_ref, w1_ref, w2_ref, b_ref, pn_ref,
                m_ref, h_o, y_o):
    agg = aggp_ref[0] + aggp_ref[1]
    z = (jnp.dot(agg * bm_ref[...], w1_ref[...],
                 preferred_element_type=jnp.float32)
         + jnp.dot(xin_ref[...], w2_ref[...],
                   preferred_element_type=jnp.float32)
         + b_ref[...])
    h = jnp.maximum(z, jnp.float32(0))
    h_o[...] = h
    y = jnp.dot(h, pn_ref[...], preferred_element_type=jnp.float32)
    y_o[...] = jnp.where(m_ref[...] > 0, y, jnp.float32(-jnp.inf))


def _tc_layer(aggp, bm, xin, w1, w2, b, pn, m):
    D = xin.shape[1]
    return pl.pallas_call(
        _layer_body,
        grid=(NBLK,),
        in_specs=[
            pl.BlockSpec((2, BN, D), lambda i: (0, i, 0)),
            pl.BlockSpec((BN, 1), lambda i: (i, 0)),
            pl.BlockSpec((BN, D), lambda i: (i, 0)),
            pl.BlockSpec((D, 32), lambda i: (0, 0)),
            pl.BlockSpec((D, 32), lambda i: (0, 0)),
            pl.BlockSpec((1, 32), lambda i: (0, 0)),
            pl.BlockSpec((32, 1), lambda i: (0, 0)),
            pl.BlockSpec((BN, 1), lambda i: (i, 0)),
        ],
        out_specs=[
            pl.BlockSpec((BN, 32), lambda i: (i, 0)),
            pl.BlockSpec((BN, 1), lambda i: (i, 0)),
        ],
        out_shape=[
            jax.ShapeDtypeStruct((N_PAD, 32), jnp.float32),
            jax.ShapeDtypeStruct((N_PAD, 1), jnp.float32),
        ],
    )(aggp, bm, xin, w1, w2, b, pn, m)


def _topk_body(k, y_ref, crit_o):
    u = _u32(y_ref[...])

    def bit(i, t):
        cand = t | (jnp.uint32(1) << jnp.uint32(31 - i))
        cnt = jnp.sum((u >= cand).astype(jnp.int32))
        return jnp.where(cnt >= k, cand, t)
    t = lax.fori_loop(0, 32, bit, jnp.uint32(0))

    cnt_gt = jnp.sum((u > t).astype(jnp.int32))
    r = k - cnt_gt
    tie = u == t
    idx = (lax.broadcasted_iota(jnp.int32, (392, 128), 0) * 128
           + lax.broadcasted_iota(jnp.int32, (392, 128), 1))

    def jbit(i, m):
        cand = m | (1 << (16 - i))
        f = jnp.sum((tie & (idx < cand)).astype(jnp.int32))
        return jnp.where(f < r, cand, m)
    m = lax.fori_loop(0, 17, jbit, jnp.int32(0))
    n_r = jnp.where(r > 0, m + 1, 0)

    lane = lax.broadcasted_iota(jnp.int32, (1, 128), 1)
    t_i = lax.bitcast_convert_type(t, jnp.int32)
    crit_o[...] = jnp.where(lane == 0, t_i, jnp.where(lane == 1, n_r, 0))


def _tc_topk(y2d, k):
    return pl.pallas_call(
        functools.partial(_topk_body, k),
        out_shape=jax.ShapeDtypeStruct((1, 128), jnp.int32),
    )(y2d)


def _gate_body(y_ref, h_ref, a0_ref, b0_ref, crit_ref, T_o, H_o, bm_o, m_o):
    pid = pl.program_id(0)
    t_u = lax.bitcast_convert_type(crit_ref[0, 0], jnp.uint32)
    n_r = crit_ref[0, 1]
    y = y_ref[...]
    u = _u32(y)
    rows = pid * BN + lax.broadcasted_iota(jnp.int32, (BN, 1), 0)
    sel = (u > t_u) | ((u == t_u) & (rows < n_r))
    mf = sel.astype(jnp.float32)
    g = jnp.tanh(y) * mf
    Hrow = h_ref[...] * g
    H_o[...] = Hrow
    T_o[...] = Hrow * (a0_ref[...] * mf)
    bm_o[...] = b0_ref[...] * mf
    m_o[...] = mf


def _tc_gate(y, h, a0, b0, crit):
    return pl.pallas_call(
        _gate_body,
        grid=(NBLK,),
        in_specs=[
            pl.BlockSpec((BN, 1), lambda i: (i, 0)),
            pl.BlockSpec((BN, 32), lambda i: (i, 0)),
            pl.BlockSpec((BN, 1), lambda i: (i, 0)),
            pl.BlockSpec((BN, 1), lambda i: (i, 0)),
            pl.BlockSpec((1, 128), lambda i: (0, 0)),
        ],
        out_specs=[
            pl.BlockSpec((BN, 32), lambda i: (i, 0)),
            pl.BlockSpec((BN, 32), lambda i: (i, 0)),
            pl.BlockSpec((BN, 1), lambda i: (i, 0)),
            pl.BlockSpec((BN, 1), lambda i: (i, 0)),
        ],
        out_shape=[
            jax.ShapeDtypeStruct((N_PAD, 32), jnp.float32),
            jax.ShapeDtypeStruct((N_PAD, 32), jnp.float32),
            jax.ShapeDtypeStruct((N_PAD, 1), jnp.float32),
            jax.ShapeDtypeStruct((N_PAD, 1), jnp.float32),
        ],
    )(y, h, a0, b0, crit)


def _final_body(aggp_ref, bm_ref, H_ref, m_ref, w1_ref, w2_ref, b_ref,
                wd_ref, bd_ref, out_o, acc):
    pid = pl.program_id(0)

    @pl.when(pid == 0)
    def _():
        acc[...] = jnp.zeros((1, 32), jnp.float32)

    agg = aggp_ref[0] + aggp_ref[1]
    z = (jnp.dot(agg * bm_ref[...], w1_ref[...],
                 preferred_element_type=jnp.float32)
         + jnp.dot(H_ref[...], w2_ref[...],
                   preferred_element_type=jnp.float32)
         + b_ref[...])
    h2 = jnp.maximum(z, jnp.float32(0)) * m_ref[...]
    acc[...] += jnp.sum(h2, axis=0, keepdims=True)

    @pl.when(pid == NBLK - 1)
    def _():
        pooled = acc[...] / jnp.float32(K2)
        logits = (jnp.dot(pooled, wd_ref[...],
                          preferred_element_type=jnp.float32) + bd_ref[...])
        mx = jnp.max(logits, axis=-1, keepdims=True)
        e = jnp.exp(logits - mx)
        out_o[...] = e / jnp.sum(e, axis=-1, keepdims=True)


def _tc_final(aggp, bm, H, m, w1, w2, b, wd, bd):
    return pl.pallas_call(
        _final_body,
        grid=(NBLK,),
        in_specs=[
            pl.BlockSpec((2, BN, 32), lambda i: (0, i, 0)),
            pl.BlockSpec((BN, 1), lambda i: (i, 0)),
            pl.BlockSpec((BN, 32), lambda i: (i, 0)),
            pl.BlockSpec((BN, 1), lambda i: (i, 0)),
            pl.BlockSpec((32, 32), lambda i: (0, 0)),
            pl.BlockSpec((32, 32), lambda i: (0, 0)),
            pl.BlockSpec((1, 32), lambda i: (0, 0)),
            pl.BlockSpec((32, 3), lambda i: (0, 0)),
            pl.BlockSpec((1, 3), lambda i: (0, 0)),
        ],
        out_specs=pl.BlockSpec((1, 3), lambda i: (0, 0)),
        out_shape=jax.ShapeDtypeStruct((1, 3), jnp.float32),
        scratch_shapes=[pltpu.VMEM((1, 32), jnp.float32)],
    )(aggp, bm, H, m, w1, w2, b, wd, bd)


def kernel(x, W1a, W2a, ba, p, W1b, W2b, bb, W1c, W2c, bc, Wd, bd,
           edge_index, i):
    f32 = jnp.float32
    src = edge_index[0]
    dst = edge_index[1]
    pad = jnp.full((E_PAD - E,), TRASH, jnp.int32)
    srcr = jnp.concatenate([src, pad]).reshape(NW, GRP * NGRP, CHUNK)
    dstr = jnp.concatenate([dst, pad]).reshape(NW, GRP * NGRP, CHUNK)

    x4 = jnp.zeros((N_PAD, 4), f32).at[:N, :3].set(x)
    W1a4 = jnp.zeros((4, 32), f32).at[:3].set(W1a)
    W2a4 = jnp.zeros((4, 32), f32).at[:3].set(W2a)
    pn = (p / jnp.linalg.norm(p)).reshape(32, 1)
    valid = (jnp.arange(N_PAD) < N).astype(f32).reshape(N_PAD, 1)
    zeros1 = jnp.zeros((N_PAD,), f32)
    zeros4 = jnp.zeros((N_PAD, 4), f32)
    zeros32 = jnp.zeros((N_PAD, 32), f32)

    deg = _sc_deg(srcr, dstr, zeros1)
    a0, b0, T0 = _tc_deg(deg, x4)
    agg0 = _sc_agg4(srcr, dstr, T0, zeros4)
    h, y1 = _tc_layer(agg0, b0, x4, W1a4, W2a4, ba.reshape(1, 32), pn, valid)
    crit1 = _tc_topk(y1.reshape(392, 128), K1)
    T1, H1, bm1, m1 = _tc_gate(y1, h, a0, b0, crit1)
    agg1 = _sc_agg32(srcr, dstr, T1, zeros32)
    h1, y2 = _tc_layer(agg1, bm1, H1, W1b, W2b, bb.reshape(1, 32), pn, m1)
    crit2 = _tc_topk(y2.reshape(392, 128), K2)
    T2, H2, bm2, m2 = _tc_gate(y2, h1, a0, b0, crit2)
    agg2 = _sc_agg32(srcr, dstr, T2, zeros32)
    return _tc_final(agg2, bm2, H2, m2, W1c, W2c, bc.reshape(1, 32),
                     Wd, bd.reshape(1, 3))


# speedup vs baseline: 41.4422x; 41.4422x over previous
import functools

import jax
import jax.numpy as jnp
from jax import lax
from jax.experimental import pallas as pl
from jax.experimental.pallas import tpu as pltpu
from jax.experimental.pallas import tpu_sc as plsc

N = 50000
E = 1600000
N_PAD = 50176
NW = 32
CHUNK = 128
GRP = 8
NGRP = 49
E_PER_TILE = CHUNK * GRP * NGRP
E_PAD = NW * E_PER_TILE
ROWS_PER_TILE = N_PAD // 16
CP = 392
TRASH = N
K1 = 25000
K2 = 12500
BN = 1024
NBLK = N_PAD // BN

@functools.cache
def _mesh():
    return plsc.VectorSubcoreMesh(core_axis_name="c", subcore_axis_name="s")


def _deg_body(srcr, dstr, zeros1, outS0, outS1, outD0, outD1,
              degS, degD, ones_v, idx_v, buf_v):
    c = lax.axis_index("c")
    s = lax.axis_index("s")
    w = c * 16 + s

    def setones(j, _):
        ones_v[pl.ds(j * 16, 16)] = jnp.ones((16,), jnp.float32)
        return 0
    lax.fori_loop(0, CHUNK // 16, setones, 0)

    row0 = s * ROWS_PER_TILE
    pltpu.sync_copy(zeros1.at[pl.ds(0, CP)], buf_v)

    def zchunk(t, _):
        pltpu.sync_copy(buf_v, degS.at[pl.ds(row0 + t * CP, CP)])
        pltpu.sync_copy(buf_v, degD.at[pl.ds(row0 + t * CP, CP)])
        return 0
    lax.fori_loop(0, ROWS_PER_TILE // CP, zchunk, 0)
    plsc.subcore_barrier()

    def grp(g, _):
        pltpu.sync_copy(srcr.at[w, pl.ds(g * GRP, GRP)], idx_v)

        def chunk_s(j, _):
            pltpu.sync_copy(ones_v, degS.at[idx_v.at[j]], add=True)
            return 0
        lax.fori_loop(0, GRP, chunk_s, 0)
        pltpu.sync_copy(dstr.at[w, pl.ds(g * GRP, GRP)], idx_v)

        def chunk_d(j, _):
            pltpu.sync_copy(ones_v, degD.at[idx_v.at[j]], add=True)
            return 0
        lax.fori_loop(0, GRP, chunk_d, 0)
        return 0
    lax.fori_loop(0, NGRP, grp, 0)

    plsc.subcore_barrier()

    def out1(acc, dstref):
        def cp(t, _):
            sl = pl.ds(row0 + t * CP, CP)
            pltpu.sync_copy(acc.at[sl], buf_v)
            pltpu.sync_copy(buf_v, dstref.at[sl])
            return 0
        lax.fori_loop(0, ROWS_PER_TILE // CP, cp, 0)

    @pl.when(c == 0)
    def _():
        out1(degS, outS0)
        out1(degD, outD0)

    @pl.when(c == 1)
    def _():
        out1(degS, outS1)
        out1(degD, outD1)


@functools.cache
def _sc_deg_fn():
    return pl.kernel(
        _deg_body,
        out_type=[jax.ShapeDtypeStruct((N_PAD,), jnp.float32)] * 4,
        mesh=_mesh(),
        compiler_params=pltpu.CompilerParams(use_tc_tiling_on_sc=False),
        scratch_types=[
            pltpu.VMEM_SHARED((N_PAD,), jnp.float32),
            pltpu.VMEM_SHARED((N_PAD,), jnp.float32),
            pltpu.VMEM((CHUNK,), jnp.float32),
            pltpu.VMEM((GRP, CHUNK), jnp.int32),
            pltpu.VMEM((CP,), jnp.float32),
        ],
    )


def _sc_deg(srcr, dstr, zeros1):
    return _sc_deg_fn()(srcr, dstr, zeros1)


def _agg_body(D, srcr, dstr, tab, zerosD, out, acc, idxs_v, idxd_v,
              rows_v, buf_v, sem):
    c = lax.axis_index("c")
    s = lax.axis_index("s")
    w = c * 16 + s

    row0 = s * ROWS_PER_TILE
    pltpu.sync_copy(zerosD.at[pl.ds(0, CP)], buf_v)

    def zchunk(t, _):
        pltpu.sync_copy(buf_v, acc.at[pl.ds(row0 + t * CP, CP)])
        return 0
    lax.fori_loop(0, ROWS_PER_TILE // CP, zchunk, 0)
    plsc.subcore_barrier()

    def grp(g, _):
        pltpu.sync_copy(srcr.at[w, pl.ds(g * GRP, GRP)], idxs_v)
        pltpu.sync_copy(dstr.at[w, pl.ds(g * GRP, GRP)], idxd_v)

        def chunk(j, _):
            pltpu.async_copy(tab.at[idxs_v.at[j]], rows_v, sem).wait()
            pltpu.sync_copy(rows_v, acc.at[idxd_v.at[j]], add=True)
            return 0
        lax.fori_loop(0, GRP, chunk, 0)
        return 0
    lax.fori_loop(0, NGRP, grp, 0)

    plsc.subcore_barrier()

    def cp(t, _):
        sl = pl.ds(row0 + t * CP, CP)
        pltpu.sync_copy(acc.at[sl], buf_v)
        pltpu.sync_copy(buf_v, out.at[c, sl])
        return 0
    lax.fori_loop(0, ROWS_PER_TILE // CP, cp, 0)


@functools.cache
def _sc_agg_fn(D):
    return pl.kernel(
        functools.partial(_agg_body, D),
        out_type=jax.ShapeDtypeStruct((2, N_PAD, D), jnp.float32),
        mesh=_mesh(),
        compiler_params=pltpu.CompilerParams(use_tc_tiling_on_sc=False),
        scratch_types=[
            pltpu.VMEM_SHARED((N_PAD, D), jnp.float32),
            pltpu.VMEM((GRP, CHUNK), jnp.int32),
            pltpu.VMEM((GRP, CHUNK), jnp.int32),
            pltpu.VMEM((CHUNK, D), jnp.float32),
            pltpu.VMEM((CP, D), jnp.float32),
            pltpu.SemaphoreType.DMA,
        ],
    )


def _sc_agg(srcr, dstr, tab, zerosD):
    return _sc_agg_fn(tab.shape[1])(srcr, dstr, tab, zerosD)


def _u32(y):
    u = lax.bitcast_convert_type(y, jnp.uint32)
    return jnp.where((u >> jnp.uint32(31)) > jnp.uint32(0),
                     ~u, u | jnp.uint32(0x80000000))


def _deg_tc_body(s0_ref, s1_ref, d0_ref, d1_ref, x4_ref, a0_o, b0_o, t0_o):
    ds = s0_ref[...] + s1_ref[...]
    dd = d0_ref[...] + d1_ref[...]
    a0 = jnp.where(ds > 0, lax.rsqrt(jnp.maximum(ds, 1e-12)),
                   jnp.float32(0))
    b0 = jnp.where(dd > 0, lax.rsqrt(jnp.maximum(dd, 1e-12)),
                   jnp.float32(0))
    a0_o[...] = a0
    b0_o[...] = b0
    t0_o[...] = x4_ref[...] * a0


def _tc_deg(s0, s1, d0, d1, x4):
    return pl.pallas_call(
        _deg_tc_body,
        grid=(NBLK,),
        in_specs=[
            pl.BlockSpec((BN, 1), lambda i: (i, 0)),
            pl.BlockSpec((BN, 1), lambda i: (i, 0)),
            pl.BlockSpec((BN, 1), lambda i: (i, 0)),
            pl.BlockSpec((BN, 1), lambda i: (i, 0)),
            pl.BlockSpec((BN, 4), lambda i: (i, 0)),
        ],
        out_specs=[
            pl.BlockSpec((BN, 1), lambda i: (i, 0)),
            pl.BlockSpec((BN, 1), lambda i: (i, 0)),
            pl.BlockSpec((BN, 4), lambda i: (i, 0)),
        ],
        out_shape=[
            jax.ShapeDtypeStruct((N_PAD, 1), jnp.float32),
            jax.ShapeDtypeStruct((N_PAD, 1), jnp.float32),
            jax.ShapeDtypeStruct((N_PAD, 4), jnp.float32),
        ],
    )(s0.reshape(N_PAD, 1), s1.reshape(N_PAD, 1),
      d0.reshape(N_PAD, 1), d1.reshape(N_PAD, 1), x4)


def _layer_z(aggs, w1s, bm_ref, xin_ref, w2_ref, b_ref):
    z = (jnp.dot(xin_ref[...], w2_ref[...],
                 preferred_element_type=jnp.float32) + b_ref[...])
    for ap, w1 in zip(aggs, w1s):
        agg = ap[0] + ap[1]
        z = z + jnp.dot(agg * bm_ref[...], w1[...],
                        preferred_element_type=jnp.float32)
    return z


def _layer_body(nparts, *refs):
    aggs = refs[:nparts]
    bm_ref, xin_ref = refs[nparts:nparts + 2]
    w1s = refs[nparts + 2:2 * nparts + 2]
    w2_ref, b_ref, pn_ref, m_ref, h_o, y_o = refs[2 * nparts + 2:]
    h = jnp.maximum(_layer_z(aggs, w1s, bm_ref, xin_ref, w2_ref, b_ref),
                    jnp.float32(0))
    h_o[...] = h
    y = jnp.dot(h, pn_ref[...], preferred_element_type=jnp.float32)
    y_o[...] = jnp.where(m_ref[...] > 0, y, jnp.float32(-jnp.inf))


def _tc_layer(aggs, bm, xin, w1s, w2, b, pn, m):
    D = xin.shape[1]
    n = len(aggs)
    agg_specs = [pl.BlockSpec((2, BN, a.shape[2]), lambda i: (0, i, 0))
                 for a in aggs]
    w1_specs = [pl.BlockSpec(w.shape, lambda i: (0, 0)) for w in w1s]
    return pl.pallas_call(
        functools.partial(_layer_body, n),
        grid=(NBLK,),
        in_specs=agg_specs + [
            pl.BlockSpec((BN, 1), lambda i: (i, 0)),
            pl.BlockSpec((BN, D), lambda i: (i, 0)),
        ] + w1_specs + [
            pl.BlockSpec((D, 32), lambda i: (0, 0)),
            pl.BlockSpec((1, 32), lambda i: (0, 0)),
            pl.BlockSpec((32, 1), lambda i: (0, 0)),
            pl.BlockSpec((BN, 1), lambda i: (i, 0)),
        ],
        out_specs=[
            pl.BlockSpec((BN, 32), lambda i: (i, 0)),
            pl.BlockSpec((BN, 1), lambda i: (i, 0)),
        ],
        out_shape=[
            jax.ShapeDtypeStruct((N_PAD, 32), jnp.float32),
            jax.ShapeDtypeStruct((N_PAD, 1), jnp.float32),
        ],
    )(*aggs, bm, xin, *w1s, w2, b, pn, m)


def _topk_body(k, y_ref, crit_o):
    u = _u32(y_ref[...])

    def bit(i, t):
        cand = t | (jnp.uint32(1) << jnp.uint32(31 - i))
        cnt = jnp.sum((u >= cand).astype(jnp.int32))
        return jnp.where(cnt >= k, cand, t)
    t = lax.fori_loop(0, 32, bit, jnp.uint32(0))

    cnt_gt = jnp.sum((u > t).astype(jnp.int32))
    r = k - cnt_gt
    tie = u == t
    idx = (lax.broadcasted_iota(jnp.int32, (392, 128), 0) * 128
           + lax.broadcasted_iota(jnp.int32, (392, 128), 1))

    def jbit(i, m):
        cand = m | (1 << (16 - i))
        f = jnp.sum((tie & (idx < cand)).astype(jnp.int32))
        return jnp.where(f < r, cand, m)
    m = lax.fori_loop(0, 17, jbit, jnp.int32(0))
    n_r = jnp.where(r > 0, m + 1, 0)

    lane = lax.broadcasted_iota(jnp.int32, (1, 128), 1)
    t_i = lax.bitcast_convert_type(t, jnp.int32)
    crit_o[...] = jnp.where(lane == 0, t_i, jnp.where(lane == 1, n_r, 0))


def _tc_topk(y2d, k):
    return pl.pallas_call(
        functools.partial(_topk_body, k),
        out_shape=jax.ShapeDtypeStruct((1, 128), jnp.int32),
    )(y2d)


def _gate_body(y_ref, h_ref, a0_ref, b0_ref, crit_ref, T_o, H_o, bm_o, m_o):
    pid = pl.program_id(0)
    t_u = lax.bitcast_convert_type(crit_ref[0, 0], jnp.uint32)
    n_r = crit_ref[0, 1]
    y = y_ref[...]
    u = _u32(y)
    rows = pid * BN + lax.broadcasted_iota(jnp.int32, (BN, 1), 0)
    sel = (u > t_u) | ((u == t_u) & (rows < n_r))
    mf = sel.astype(jnp.float32)
    g = jnp.tanh(y) * mf
    Hrow = h_ref[...] * g
    H_o[...] = Hrow
    T_o[...] = Hrow * (a0_ref[...] * mf)
    bm_o[...] = b0_ref[...] * mf
    m_o[...] = mf


def _tc_gate(y, h, a0, b0, crit):
    return pl.pallas_call(
        _gate_body,
        grid=(NBLK,),
        in_specs=[
            pl.BlockSpec((BN, 1), lambda i: (i, 0)),
            pl.BlockSpec((BN, 32), lambda i: (i, 0)),
            pl.BlockSpec((BN, 1), lambda i: (i, 0)),
            pl.BlockSpec((BN, 1), lambda i: (i, 0)),
            pl.BlockSpec((1, 128), lambda i: (0, 0)),
        ],
        out_specs=[
            pl.BlockSpec((BN, 32), lambda i: (i, 0)),
            pl.BlockSpec((BN, 32), lambda i: (i, 0)),
            pl.BlockSpec((BN, 1), lambda i: (i, 0)),
            pl.BlockSpec((BN, 1), lambda i: (i, 0)),
        ],
        out_shape=[
            jax.ShapeDtypeStruct((N_PAD, 32), jnp.float32),
            jax.ShapeDtypeStruct((N_PAD, 32), jnp.float32),
            jax.ShapeDtypeStruct((N_PAD, 1), jnp.float32),
            jax.ShapeDtypeStruct((N_PAD, 1), jnp.float32),
        ],
    )(y, h, a0, b0, crit)


def _final_body(nparts, *refs):
    aggs = refs[:nparts]
    bm_ref, H_ref = refs[nparts:nparts + 2]
    w1s = refs[nparts + 2:2 * nparts + 2]
    m_ref, w2_ref, b_ref, wd_ref, bd_ref, out_o, acc = refs[2 * nparts + 2:]
    pid = pl.program_id(0)

    @pl.when(pid == 0)
    def _():
        acc[...] = jnp.zeros((1, 32), jnp.float32)

    z = _layer_z(aggs, w1s, bm_ref, H_ref, w2_ref, b_ref)
    h2 = jnp.maximum(z, jnp.float32(0)) * m_ref[...]
    acc[...] += jnp.sum(h2, axis=0, keepdims=True)

    @pl.when(pid == NBLK - 1)
    def _():
        pooled = acc[...] / jnp.float32(K2)
        logits = (jnp.dot(pooled, wd_ref[...],
                          preferred_element_type=jnp.float32) + bd_ref[...])
        mx = jnp.max(logits, axis=-1, keepdims=True)
        e = jnp.exp(logits - mx)
        out_o[...] = e / jnp.sum(e, axis=-1, keepdims=True)


def _tc_final(aggs, bm, H, w1s, m, w2, b, wd, bd):
    n = len(aggs)
    agg_specs = [pl.BlockSpec((2, BN, a.shape[2]), lambda i: (0, i, 0))
                 for a in aggs]
    w1_specs = [pl.BlockSpec(w.shape, lambda i: (0, 0)) for w in w1s]
    return pl.pallas_call(
        functools.partial(_final_body, n),
        grid=(NBLK,),
        in_specs=agg_specs + [
            pl.BlockSpec((BN, 1), lambda i: (i, 0)),
            pl.BlockSpec((BN, 32), lambda i: (i, 0)),
        ] + w1_specs + [
            pl.BlockSpec((BN, 1), lambda i: (i, 0)),
            pl.BlockSpec((32, 32), lambda i: (0, 0)),
            pl.BlockSpec((1, 32), lambda i: (0, 0)),
            pl.BlockSpec((32, 3), lambda i: (0, 0)),
            pl.BlockSpec((1, 3), lambda i: (0, 0)),
        ],
        out_specs=pl.BlockSpec((1, 3), lambda i: (0, 0)),
        out_shape=jax.ShapeDtypeStruct((1, 3), jnp.float32),
        scratch_shapes=[pltpu.VMEM((1, 32), jnp.float32)],
    )(*aggs, bm, H, *w1s, m, w2, b, wd, bd)


def kernel(x, W1a, W2a, ba, p, W1b, W2b, bb, W1c, W2c, bc, Wd, bd,
           edge_index, i):
    f32 = jnp.float32
    src = edge_index[0]
    dst = edge_index[1]
    pad = jnp.full((E_PAD - E,), TRASH, jnp.int32)
    srcr = jnp.concatenate([src, pad]).reshape(NW, GRP * NGRP, CHUNK)
    dstr = jnp.concatenate([dst, pad]).reshape(NW, GRP * NGRP, CHUNK)

    x4 = jnp.zeros((N_PAD, 4), f32).at[:N, :3].set(x)
    W1a4 = jnp.zeros((4, 32), f32).at[:3].set(W1a)
    W2a4 = jnp.zeros((4, 32), f32).at[:3].set(W2a)
    pn = (p / jnp.linalg.norm(p)).reshape(32, 1)
    valid = (jnp.arange(N_PAD) < N).astype(f32).reshape(N_PAD, 1)
    zeros1 = jnp.zeros((N_PAD,), f32)
    zeros4 = jnp.zeros((N_PAD, 4), f32)
    zeros32 = jnp.zeros((N_PAD, 32), f32)

    s0, s1, d0, d1 = _sc_deg(srcr, dstr, zeros1)
    a0, b0, T0 = _tc_deg(s0, s1, d0, d1, x4)
    agg0 = _sc_agg(srcr, dstr, T0, zeros4)
    h, y1 = _tc_layer([agg0], b0, x4, [W1a4], W2a4, ba.reshape(1, 32), pn,
                      valid)
    crit1 = _tc_topk(y1.reshape(392, 128), K1)
    T1, H1, bm1, m1 = _tc_gate(y1, h, a0, b0, crit1)
    agg1 = [_sc_agg(srcr, dstr, T1, zeros32)]
    h1, y2 = _tc_layer(agg1, bm1, H1, [W1b], W2b,
                       bb.reshape(1, 32), pn, m1)
    crit2 = _tc_topk(y2.reshape(392, 128), K2)
    T2, H2, bm2, m2 = _tc_gate(y2, h1, a0, b0, crit2)
    agg2 = [_sc_agg(srcr, dstr, T2, zeros32)]
    return _tc_final(agg2, bm2, H2, [W1c], m2, W2c,
                     bc.reshape(1, 32), Wd, bd.reshape(1, 3))


# speedup vs baseline: 45.1233x; 1.0888x over previous
import functools

import jax
import jax.numpy as jnp
from jax import lax
from jax.experimental import pallas as pl
from jax.experimental.pallas import tpu as pltpu
from jax.experimental.pallas import tpu_sc as plsc

N = 50000
E = 1600000
N_PAD = 50176
NW = 32
CHUNK = 128
GRP = 8
NGRP = 49
E_PER_TILE = CHUNK * GRP * NGRP
E_PAD = NW * E_PER_TILE
ROWS_PER_TILE = N_PAD // 16
CP = 112
TRASH = N
K1 = 25000
K2 = 12500
BN = 1024
NBLK = N_PAD // BN

@functools.cache
def _mesh():
    return plsc.VectorSubcoreMesh(core_axis_name="c", subcore_axis_name="s")


def _deg_body(srcr, dstr, zeros1, outS0, outS1, outD0, outD1,
              degS, degD, ones_v, idx_v, buf_v, dsem):
    c = lax.axis_index("c")
    s = lax.axis_index("s")
    w = c * 16 + s

    def setones(j, _):
        ones_v[pl.ds(j * 16, 16)] = jnp.ones((16,), jnp.float32)
        return 0
    lax.fori_loop(0, CHUNK // 16, setones, 0)

    row0 = s * ROWS_PER_TILE
    pltpu.sync_copy(zeros1.at[pl.ds(0, CP)], buf_v)

    def zchunk(t, _):
        pltpu.sync_copy(buf_v, degS.at[pl.ds(row0 + t * CP, CP)])
        pltpu.sync_copy(buf_v, degD.at[pl.ds(row0 + t * CP, CP)])
        return 0
    lax.fori_loop(0, ROWS_PER_TILE // CP, zchunk, 0)
    plsc.subcore_barrier()

    def grp(g, _):
        pltpu.sync_copy(srcr.at[w, pl.ds(g * GRP, GRP)], idx_v)
        descs = [pltpu.async_copy(ones_v, degS.at[idx_v.at[j]], dsem,
                                  add=True) for j in range(GRP)]
        for dsc in descs:
            dsc.wait()
        pltpu.sync_copy(dstr.at[w, pl.ds(g * GRP, GRP)], idx_v)
        descs = [pltpu.async_copy(ones_v, degD.at[idx_v.at[j]], dsem,
                                  add=True) for j in range(GRP)]
        for dsc in descs:
            dsc.wait()
        return 0
    lax.fori_loop(0, NGRP, grp, 0)

    plsc.subcore_barrier()

    def out1(acc, dstref):
        def cp(t, _):
            sl = pl.ds(row0 + t * CP, CP)
            pltpu.sync_copy(acc.at[sl], buf_v)
            pltpu.sync_copy(buf_v, dstref.at[sl])
            return 0
        lax.fori_loop(0, ROWS_PER_TILE // CP, cp, 0)

    @pl.when(c == 0)
    def _():
        out1(degS, outS0)
        out1(degD, outD0)

    @pl.when(c == 1)
    def _():
        out1(degS, outS1)
        out1(degD, outD1)


@functools.cache
def _sc_deg_fn():
    return pl.kernel(
        _deg_body,
        out_type=[jax.ShapeDtypeStruct((N_PAD,), jnp.float32)] * 4,
        mesh=_mesh(),
        compiler_params=pltpu.CompilerParams(use_tc_tiling_on_sc=False),
        scratch_types=[
            pltpu.VMEM_SHARED((N_PAD,), jnp.float32),
            pltpu.VMEM_SHARED((N_PAD,), jnp.float32),
            pltpu.VMEM((CHUNK,), jnp.float32),
            pltpu.VMEM((GRP, CHUNK), jnp.int32),
            pltpu.VMEM((CP,), jnp.float32),
            pltpu.SemaphoreType.DMA,
        ],
    )


def _sc_deg(srcr, dstr, zeros1):
    return _sc_deg_fn()(srcr, dstr, zeros1)


def _agg_body(D, srcr, dstr, tab, zerosD, out, acc, idxs_v, idxd_v,
              r0_v, r1_v, buf_v, gs0, gs1, ss0, ss1):
    c = lax.axis_index("c")
    s = lax.axis_index("s")
    w = c * 16 + s

    row0 = s * ROWS_PER_TILE
    pltpu.sync_copy(zerosD.at[pl.ds(0, CP)], buf_v)

    def zchunk(t, _):
        pltpu.sync_copy(buf_v, acc.at[pl.ds(row0 + t * CP, CP)])
        return 0
    lax.fori_loop(0, ROWS_PER_TILE // CP, zchunk, 0)
    plsc.subcore_barrier()

    rows = (r0_v, r1_v)
    gsem = (gs0, gs1)
    ssem = (ss0, ss1)

    def grp(g, _):
        pltpu.sync_copy(srcr.at[w, pl.ds(g * GRP, GRP)], idxs_v)
        pltpu.sync_copy(dstr.at[w, pl.ds(g * GRP, GRP)], idxd_v)
        gd = [None, None]
        sd = [None, None]
        gd[0] = pltpu.async_copy(tab.at[idxs_v.at[0]], rows[0], gsem[0])
        for j in range(GRP):
            b = j % 2
            gd[b].wait()
            if j >= 1:
                sd[1 - b].wait()
            sd[b] = pltpu.async_copy(rows[b], acc.at[idxd_v.at[j]],
                                     ssem[b], add=True)
            if j + 1 < GRP:
                gd[1 - b] = pltpu.async_copy(tab.at[idxs_v.at[j + 1]],
                                             rows[1 - b], gsem[1 - b])
        sd[(GRP - 1) % 2].wait()
        return 0
    lax.fori_loop(0, NGRP, grp, 0)

    plsc.subcore_barrier()

    def cp(t, _):
        sl = pl.ds(row0 + t * CP, CP)
        pltpu.sync_copy(acc.at[sl], buf_v)
        pltpu.sync_copy(buf_v, out.at[c, sl])
        return 0
    lax.fori_loop(0, ROWS_PER_TILE // CP, cp, 0)


@functools.cache
def _sc_agg_fn(D):
    return pl.kernel(
        functools.partial(_agg_body, D),
        out_type=jax.ShapeDtypeStruct((2, N_PAD, D), jnp.float32),
        mesh=_mesh(),
        compiler_params=pltpu.CompilerParams(use_tc_tiling_on_sc=False),
        scratch_types=[
            pltpu.VMEM_SHARED((N_PAD, D), jnp.float32),
            pltpu.VMEM((GRP, CHUNK), jnp.int32),
            pltpu.VMEM((GRP, CHUNK), jnp.int32),
            pltpu.VMEM((CHUNK, D), jnp.float32),
            pltpu.VMEM((CHUNK, D), jnp.float32),
            pltpu.VMEM((CP, D), jnp.float32),
            pltpu.SemaphoreType.DMA,
            pltpu.SemaphoreType.DMA,
            pltpu.SemaphoreType.DMA,
            pltpu.SemaphoreType.DMA,
        ],
    )


def _sc_agg(srcr, dstr, tab, zerosD):
    return _sc_agg_fn(tab.shape[1])(srcr, dstr, tab, zerosD)


def _u32(y):
    u = lax.bitcast_convert_type(y, jnp.uint32)
    return jnp.where((u >> jnp.uint32(31)) > jnp.uint32(0),
                     ~u, u | jnp.uint32(0x80000000))


def _deg_tc_body(s0_ref, s1_ref, d0_ref, d1_ref, x4_ref, a0_o, b0_o, t0_o):
    ds = s0_ref[...] + s1_ref[...]
    dd = d0_ref[...] + d1_ref[...]
    a0 = jnp.where(ds > 0, lax.rsqrt(jnp.maximum(ds, 1e-12)),
                   jnp.float32(0))
    b0 = jnp.where(dd > 0, lax.rsqrt(jnp.maximum(dd, 1e-12)),
                   jnp.float32(0))
    a0_o[...] = a0
    b0_o[...] = b0
    t0_o[...] = x4_ref[...] * a0


def _tc_deg(s0, s1, d0, d1, x4):
    return pl.pallas_call(
        _deg_tc_body,
        grid=(NBLK,),
        in_specs=[
            pl.BlockSpec((BN, 1), lambda i: (i, 0)),
            pl.BlockSpec((BN, 1), lambda i: (i, 0)),
            pl.BlockSpec((BN, 1), lambda i: (i, 0)),
            pl.BlockSpec((BN, 1), lambda i: (i, 0)),
            pl.BlockSpec((BN, 4), lambda i: (i, 0)),
        ],
        out_specs=[
            pl.BlockSpec((BN, 1), lambda i: (i, 0)),
            pl.BlockSpec((BN, 1), lambda i: (i, 0)),
            pl.BlockSpec((BN, 4), lambda i: (i, 0)),
        ],
        out_shape=[
            jax.ShapeDtypeStruct((N_PAD, 1), jnp.float32),
            jax.ShapeDtypeStruct((N_PAD, 1), jnp.float32),
            jax.ShapeDtypeStruct((N_PAD, 4), jnp.float32),
        ],
    )(s0.reshape(N_PAD, 1), s1.reshape(N_PAD, 1),
      d0.reshape(N_PAD, 1), d1.reshape(N_PAD, 1), x4)


def _layer_z(aggs, w1s, bm_ref, xin_ref, w2_ref, b_ref):
    z = (jnp.dot(xin_ref[...], w2_ref[...],
                 preferred_element_type=jnp.float32) + b_ref[...])
    for ap, w1 in zip(aggs, w1s):
        agg = ap[0] + ap[1]
        z = z + jnp.dot(agg * bm_ref[...], w1[...],
                        preferred_element_type=jnp.float32)
    return z


def _layer_body(nparts, *refs):
    aggs = refs[:nparts]
    bm_ref, xin_ref = refs[nparts:nparts + 2]
    w1s = refs[nparts + 2:2 * nparts + 2]
    w2_ref, b_ref, pn_ref, m_ref, h_o, y_o = refs[2 * nparts + 2:]
    h = jnp.maximum(_layer_z(aggs, w1s, bm_ref, xin_ref, w2_ref, b_ref),
                    jnp.float32(0))
    h_o[...] = h
    y = jnp.dot(h, pn_ref[...], preferred_element_type=jnp.float32)
    y_o[...] = jnp.where(m_ref[...] > 0, y, jnp.float32(-jnp.inf))


def _tc_layer(aggs, bm, xin, w1s, w2, b, pn, m):
    D = xin.shape[1]
    n = len(aggs)
    agg_specs = [pl.BlockSpec((2, BN, a.shape[2]), lambda i: (0, i, 0))
                 for a in aggs]
    w1_specs = [pl.BlockSpec(w.shape, lambda i: (0, 0)) for w in w1s]
    return pl.pallas_call(
        functools.partial(_layer_body, n),
        grid=(NBLK,),
        in_specs=agg_specs + [
            pl.BlockSpec((BN, 1), lambda i: (i, 0)),
            pl.BlockSpec((BN, D), lambda i: (i, 0)),
        ] + w1_specs + [
            pl.BlockSpec((D, 32), lambda i: (0, 0)),
            pl.BlockSpec((1, 32), lambda i: (0, 0)),
            pl.BlockSpec((32, 1), lambda i: (0, 0)),
            pl.BlockSpec((BN, 1), lambda i: (i, 0)),
        ],
        out_specs=[
            pl.BlockSpec((BN, 32), lambda i: (i, 0)),
            pl.BlockSpec((BN, 1), lambda i: (i, 0)),
        ],
        out_shape=[
            jax.ShapeDtypeStruct((N_PAD, 32), jnp.float32),
            jax.ShapeDtypeStruct((N_PAD, 1), jnp.float32),
        ],
    )(*aggs, bm, xin, *w1s, w2, b, pn, m)


def _topk_body(k, y_ref, crit_o):
    u = _u32(y_ref[...])

    def bit(i, t):
        cand = t | (jnp.uint32(1) << jnp.uint32(31 - i))
        cnt = jnp.sum((u >= cand).astype(jnp.int32))
        return jnp.where(cnt >= k, cand, t)
    t = lax.fori_loop(0, 32, bit, jnp.uint32(0))

    cnt_gt = jnp.sum((u > t).astype(jnp.int32))
    r = k - cnt_gt
    tie = u == t
    idx = (lax.broadcasted_iota(jnp.int32, (392, 128), 0) * 128
           + lax.broadcasted_iota(jnp.int32, (392, 128), 1))

    def jbit(i, m):
        cand = m | (1 << (16 - i))
        f = jnp.sum((tie & (idx < cand)).astype(jnp.int32))
        return jnp.where(f < r, cand, m)
    m = lax.fori_loop(0, 17, jbit, jnp.int32(0))
    n_r = jnp.where(r > 0, m + 1, 0)

    lane = lax.broadcasted_iota(jnp.int32, (1, 128), 1)
    t_i = lax.bitcast_convert_type(t, jnp.int32)
    crit_o[...] = jnp.where(lane == 0, t_i, jnp.where(lane == 1, n_r, 0))


def _tc_topk(y2d, k):
    return pl.pallas_call(
        functools.partial(_topk_body, k),
        out_shape=jax.ShapeDtypeStruct((1, 128), jnp.int32),
    )(y2d)


def _gate_body(y_ref, h_ref, a0_ref, b0_ref, crit_ref, T_o, H_o, bm_o, m_o):
    pid = pl.program_id(0)
    t_u = lax.bitcast_convert_type(crit_ref[0, 0], jnp.uint32)
    n_r = crit_ref[0, 1]
    y = y_ref[...]
    u = _u32(y)
    rows = pid * BN + lax.broadcasted_iota(jnp.int32, (BN, 1), 0)
    sel = (u > t_u) | ((u == t_u) & (rows < n_r))
    mf = sel.astype(jnp.float32)
    g = jnp.tanh(y) * mf
    Hrow = h_ref[...] * g
    H_o[...] = Hrow
    T_o[...] = Hrow * (a0_ref[...] * mf)
    bm_o[...] = b0_ref[...] * mf
    m_o[...] = mf


def _tc_gate(y, h, a0, b0, crit):
    return pl.pallas_call(
        _gate_body,
        grid=(NBLK,),
        in_specs=[
            pl.BlockSpec((BN, 1), lambda i: (i, 0)),
            pl.BlockSpec((BN, 32), lambda i: (i, 0)),
            pl.BlockSpec((BN, 1), lambda i: (i, 0)),
            pl.BlockSpec((BN, 1), lambda i: (i, 0)),
            pl.BlockSpec((1, 128), lambda i: (0, 0)),
        ],
        out_specs=[
            pl.BlockSpec((BN, 32), lambda i: (i, 0)),
            pl.BlockSpec((BN, 32), lambda i: (i, 0)),
            pl.BlockSpec((BN, 1), lambda i: (i, 0)),
            pl.BlockSpec((BN, 1), lambda i: (i, 0)),
        ],
        out_shape=[
            jax.ShapeDtypeStruct((N_PAD, 32), jnp.float32),
            jax.ShapeDtypeStruct((N_PAD, 32), jnp.float32),
            jax.ShapeDtypeStruct((N_PAD, 1), jnp.float32),
            jax.ShapeDtypeStruct((N_PAD, 1), jnp.float32),
        ],
    )(y, h, a0, b0, crit)


def _final_body(nparts, *refs):
    aggs = refs[:nparts]
    bm_ref, H_ref = refs[nparts:nparts + 2]
    w1s = refs[nparts + 2:2 * nparts + 2]
    m_ref, w2_ref, b_ref, wd_ref, bd_ref, out_o, acc = refs[2 * nparts + 2:]
    pid = pl.program_id(0)

    @pl.when(pid == 0)
    def _():
        acc[...] = jnp.zeros((1, 32), jnp.float32)

    z = _layer_z(aggs, w1s, bm_ref, H_ref, w2_ref, b_ref)
    h2 = jnp.maximum(z, jnp.float32(0)) * m_ref[...]
    acc[...] += jnp.sum(h2, axis=0, keepdims=True)

    @pl.when(pid == NBLK - 1)
    def _():
        pooled = acc[...] / jnp.float32(K2)
        logits = (jnp.dot(pooled, wd_ref[...],
                          preferred_element_type=jnp.float32) + bd_ref[...])
        mx = jnp.max(logits, axis=-1, keepdims=True)
        e = jnp.exp(logits - mx)
        out_o[...] = e / jnp.sum(e, axis=-1, keepdims=True)


def _tc_final(aggs, bm, H, w1s, m, w2, b, wd, bd):
    n = len(aggs)
    agg_specs = [pl.BlockSpec((2, BN, a.shape[2]), lambda i: (0, i, 0))
                 for a in aggs]
    w1_specs = [pl.BlockSpec(w.shape, lambda i: (0, 0)) for w in w1s]
    return pl.pallas_call(
        functools.partial(_final_body, n),
        grid=(NBLK,),
        in_specs=agg_specs + [
            pl.BlockSpec((BN, 1), lambda i: (i, 0)),
            pl.BlockSpec((BN, 32), lambda i: (i, 0)),
        ] + w1_specs + [
            pl.BlockSpec((BN, 1), lambda i: (i, 0)),
            pl.BlockSpec((32, 32), lambda i: (0, 0)),
            pl.BlockSpec((1, 32), lambda i: (0, 0)),
            pl.BlockSpec((32, 3), lambda i: (0, 0)),
            pl.BlockSpec((1, 3), lambda i: (0, 0)),
        ],
        out_specs=pl.BlockSpec((1, 3), lambda i: (0, 0)),
        out_shape=jax.ShapeDtypeStruct((1, 3), jnp.float32),
        scratch_shapes=[pltpu.VMEM((1, 32), jnp.float32)],
    )(*aggs, bm, H, *w1s, m, w2, b, wd, bd)


def kernel(x, W1a, W2a, ba, p, W1b, W2b, bb, W1c, W2c, bc, Wd, bd,
           edge_index, i):
    f32 = jnp.float32
    src = edge_index[0]
    dst = edge_index[1]
    pad = jnp.full((E_PAD - E,), TRASH, jnp.int32)
    srcr = jnp.concatenate([src, pad]).reshape(NW, GRP * NGRP, CHUNK)
    dstr = jnp.concatenate([dst, pad]).reshape(NW, GRP * NGRP, CHUNK)

    x4 = jnp.zeros((N_PAD, 4), f32).at[:N, :3].set(x)
    W1a4 = jnp.zeros((4, 32), f32).at[:3].set(W1a)
    W2a4 = jnp.zeros((4, 32), f32).at[:3].set(W2a)
    pn = (p / jnp.linalg.norm(p)).reshape(32, 1)
    valid = (jnp.arange(N_PAD) < N).astype(f32).reshape(N_PAD, 1)
    zeros1 = jnp.zeros((N_PAD,), f32)
    zeros4 = jnp.zeros((N_PAD, 4), f32)
    zeros32 = jnp.zeros((N_PAD, 32), f32)

    s0, s1, d0, d1 = _sc_deg(srcr, dstr, zeros1)
    a0, b0, T0 = _tc_deg(s0, s1, d0, d1, x4)
    agg0 = _sc_agg(srcr, dstr, T0, zeros4)
    h, y1 = _tc_layer([agg0], b0, x4, [W1a4], W2a4, ba.reshape(1, 32), pn,
                      valid)
    crit1 = _tc_topk(y1.reshape(392, 128), K1)
    T1, H1, bm1, m1 = _tc_gate(y1, h, a0, b0, crit1)
    agg1 = [_sc_agg(srcr, dstr, T1, zeros32)]
    h1, y2 = _tc_layer(agg1, bm1, H1, [W1b], W2b,
                       bb.reshape(1, 32), pn, m1)
    crit2 = _tc_topk(y2.reshape(392, 128), K2)
    T2, H2, bm2, m2 = _tc_gate(y2, h1, a0, b0, crit2)
    agg2 = [_sc_agg(srcr, dstr, T2, zeros32)]
    return _tc_final(agg2, bm2, H2, [W1c], m2, W2c,
                     bc.reshape(1, 32), Wd, bd.reshape(1, 3))


# speedup vs baseline: 52.6083x; 1.1659x over previous
import functools

import jax
import jax.numpy as jnp
from jax import lax
from jax.experimental import pallas as pl
from jax.experimental.pallas import tpu as pltpu
from jax.experimental.pallas import tpu_sc as plsc

N = 50000
E = 1600000
N_PAD = 50176
NW = 32
CHUNK = 256
GRP = 4
NGRP = 49
E_PER_TILE = CHUNK * GRP * NGRP
E_PAD = NW * E_PER_TILE
ROWS_PER_TILE = N_PAD // 16
CP = 112
TRASH = N
K1 = 25000
K2 = 12500
BN = 1024
NBLK = N_PAD // BN

@functools.cache
def _mesh():
    return plsc.VectorSubcoreMesh(core_axis_name="c", subcore_axis_name="s")


def _deg_body(srcr, dstr, zeros1, outS0, outS1, outD0, outD1,
              degS, degD, ones_v, idx_v, buf_v, dsem):
    c = lax.axis_index("c")
    s = lax.axis_index("s")
    w = c * 16 + s

    def setones(j, _):
        ones_v[pl.ds(j * 16, 16)] = jnp.ones((16,), jnp.float32)
        return 0
    lax.fori_loop(0, CHUNK // 16, setones, 0)

    row0 = s * ROWS_PER_TILE
    pltpu.sync_copy(zeros1.at[pl.ds(0, CP)], buf_v)

    def zchunk(t, _):
        pltpu.sync_copy(buf_v, degS.at[pl.ds(row0 + t * CP, CP)])
        pltpu.sync_copy(buf_v, degD.at[pl.ds(row0 + t * CP, CP)])
        return 0
    lax.fori_loop(0, ROWS_PER_TILE // CP, zchunk, 0)
    plsc.subcore_barrier()

    def grp(g, _):
        pltpu.sync_copy(srcr.at[w, pl.ds(g * GRP, GRP)], idx_v)
        descs = [pltpu.async_copy(ones_v, degS.at[idx_v.at[j]], dsem,
                                  add=True) for j in range(GRP)]
        for dsc in descs:
            dsc.wait()
        pltpu.sync_copy(dstr.at[w, pl.ds(g * GRP, GRP)], idx_v)
        descs = [pltpu.async_copy(ones_v, degD.at[idx_v.at[j]], dsem,
                                  add=True) for j in range(GRP)]
        for dsc in descs:
            dsc.wait()
        return 0
    lax.fori_loop(0, NGRP, grp, 0)

    plsc.subcore_barrier()

    def out1(acc, dstref):
        def cp(t, _):
            sl = pl.ds(row0 + t * CP, CP)
            pltpu.sync_copy(acc.at[sl], buf_v)
            pltpu.sync_copy(buf_v, dstref.at[sl])
            return 0
        lax.fori_loop(0, ROWS_PER_TILE // CP, cp, 0)

    @pl.when(c == 0)
    def _():
        out1(degS, outS0)
        out1(degD, outD0)

    @pl.when(c == 1)
    def _():
        out1(degS, outS1)
        out1(degD, outD1)


@functools.cache
def _sc_deg_fn():
    return pl.kernel(
        _deg_body,
        out_type=[jax.ShapeDtypeStruct((N_PAD,), jnp.float32)] * 4,
        mesh=_mesh(),
        compiler_params=pltpu.CompilerParams(use_tc_tiling_on_sc=False),
        scratch_types=[
            pltpu.VMEM_SHARED((N_PAD,), jnp.float32),
            pltpu.VMEM_SHARED((N_PAD,), jnp.float32),
            pltpu.VMEM((CHUNK,), jnp.float32),
            pltpu.VMEM((GRP, CHUNK), jnp.int32),
            pltpu.VMEM((CP,), jnp.float32),
            pltpu.SemaphoreType.DMA,
        ],
    )


def _sc_deg(srcr, dstr, zeros1):
    return _sc_deg_fn()(srcr, dstr, zeros1)


def _agg_body(D, srcr, dstr, tab, zerosD, out, acc, idxs_v, idxd_v,
              r0_v, r1_v, buf_v, gs0, gs1, ss0, ss1):
    c = lax.axis_index("c")
    s = lax.axis_index("s")
    w = c * 16 + s

    row0 = s * ROWS_PER_TILE
    pltpu.sync_copy(zerosD.at[pl.ds(0, CP)], buf_v)

    def zchunk(t, _):
        pltpu.sync_copy(buf_v, acc.at[pl.ds(row0 + t * CP, CP)])
        return 0
    lax.fori_loop(0, ROWS_PER_TILE // CP, zchunk, 0)
    plsc.subcore_barrier()

    rows = (r0_v, r1_v)
    gsem = (gs0, gs1)
    ssem = (ss0, ss1)

    def grp(g, _):
        pltpu.sync_copy(srcr.at[w, pl.ds(g * GRP, GRP)], idxs_v)
        pltpu.sync_copy(dstr.at[w, pl.ds(g * GRP, GRP)], idxd_v)
        gd = [None, None]
        sd = [None, None]
        gd[0] = pltpu.async_copy(tab.at[idxs_v.at[0]], rows[0], gsem[0])
        for j in range(GRP):
            b = j % 2
            gd[b].wait()
            if j >= 1:
                sd[1 - b].wait()
            sd[b] = pltpu.async_copy(rows[b], acc.at[idxd_v.at[j]],
                                     ssem[b], add=True)
            if j + 1 < GRP:
                gd[1 - b] = pltpu.async_copy(tab.at[idxs_v.at[j + 1]],
                                             rows[1 - b], gsem[1 - b])
        sd[(GRP - 1) % 2].wait()
        return 0
    lax.fori_loop(0, NGRP, grp, 0)

    plsc.subcore_barrier()

    def cp(t, _):
        sl = pl.ds(row0 + t * CP, CP)
        pltpu.sync_copy(acc.at[sl], buf_v)
        pltpu.sync_copy(buf_v, out.at[c, sl])
        return 0
    lax.fori_loop(0, ROWS_PER_TILE // CP, cp, 0)


@functools.cache
def _sc_agg_fn(D):
    return pl.kernel(
        functools.partial(_agg_body, D),
        out_type=jax.ShapeDtypeStruct((2, N_PAD, D), jnp.float32),
        mesh=_mesh(),
        compiler_params=pltpu.CompilerParams(use_tc_tiling_on_sc=False),
        scratch_types=[
            pltpu.VMEM_SHARED((N_PAD, D), jnp.float32),
            pltpu.VMEM((GRP, CHUNK), jnp.int32),
            pltpu.VMEM((GRP, CHUNK), jnp.int32),
            pltpu.VMEM((CHUNK, D), jnp.float32),
            pltpu.VMEM((CHUNK, D), jnp.float32),
            pltpu.VMEM((CP, D), jnp.float32),
            pltpu.SemaphoreType.DMA,
            pltpu.SemaphoreType.DMA,
            pltpu.SemaphoreType.DMA,
            pltpu.SemaphoreType.DMA,
        ],
    )


def _sc_agg(srcr, dstr, tab, zerosD):
    return _sc_agg_fn(tab.shape[1])(srcr, dstr, tab, zerosD)


def _u32(y):
    u = lax.bitcast_convert_type(y, jnp.uint32)
    return jnp.where((u >> jnp.uint32(31)) > jnp.uint32(0),
                     ~u, u | jnp.uint32(0x80000000))


def _deg_tc_body(s0_ref, s1_ref, d0_ref, d1_ref, x4_ref, a0_o, b0_o, t0_o):
    ds = s0_ref[...] + s1_ref[...]
    dd = d0_ref[...] + d1_ref[...]
    a0 = jnp.where(ds > 0, lax.rsqrt(jnp.maximum(ds, 1e-12)),
                   jnp.float32(0))
    b0 = jnp.where(dd > 0, lax.rsqrt(jnp.maximum(dd, 1e-12)),
                   jnp.float32(0))
    a0_o[...] = a0
    b0_o[...] = b0
    t0_o[...] = x4_ref[...] * a0


def _tc_deg(s0, s1, d0, d1, x4):
    return pl.pallas_call(
        _deg_tc_body,
        grid=(NBLK,),
        in_specs=[
            pl.BlockSpec((BN, 1), lambda i: (i, 0)),
            pl.BlockSpec((BN, 1), lambda i: (i, 0)),
            pl.BlockSpec((BN, 1), lambda i: (i, 0)),
            pl.BlockSpec((BN, 1), lambda i: (i, 0)),
            pl.BlockSpec((BN, 4), lambda i: (i, 0)),
        ],
        out_specs=[
            pl.BlockSpec((BN, 1), lambda i: (i, 0)),
            pl.BlockSpec((BN, 1), lambda i: (i, 0)),
            pl.BlockSpec((BN, 4), lambda i: (i, 0)),
        ],
        out_shape=[
            jax.ShapeDtypeStruct((N_PAD, 1), jnp.float32),
            jax.ShapeDtypeStruct((N_PAD, 1), jnp.float32),
            jax.ShapeDtypeStruct((N_PAD, 4), jnp.float32),
        ],
    )(s0.reshape(N_PAD, 1), s1.reshape(N_PAD, 1),
      d0.reshape(N_PAD, 1), d1.reshape(N_PAD, 1), x4)


def _layer_z(aggs, w1s, bm_ref, xin_ref, w2_ref, b_ref):
    z = (jnp.dot(xin_ref[...], w2_ref[...],
                 preferred_element_type=jnp.float32) + b_ref[...])
    for ap, w1 in zip(aggs, w1s):
        agg = ap[0] + ap[1]
        z = z + jnp.dot(agg * bm_ref[...], w1[...],
                        preferred_element_type=jnp.float32)
    return z


def _layer_body(nparts, *refs):
    aggs = refs[:nparts]
    bm_ref, xin_ref = refs[nparts:nparts + 2]
    w1s = refs[nparts + 2:2 * nparts + 2]
    w2_ref, b_ref, pn_ref, m_ref, h_o, y_o = refs[2 * nparts + 2:]
    h = jnp.maximum(_layer_z(aggs, w1s, bm_ref, xin_ref, w2_ref, b_ref),
                    jnp.float32(0))
    h_o[...] = h
    y = jnp.dot(h, pn_ref[...], preferred_element_type=jnp.float32)
    y_o[...] = jnp.where(m_ref[...] > 0, y, jnp.float32(-jnp.inf))


def _tc_layer(aggs, bm, xin, w1s, w2, b, pn, m):
    D = xin.shape[1]
    n = len(aggs)
    agg_specs = [pl.BlockSpec((2, BN, a.shape[2]), lambda i: (0, i, 0))
                 for a in aggs]
    w1_specs = [pl.BlockSpec(w.shape, lambda i: (0, 0)) for w in w1s]
    return pl.pallas_call(
        functools.partial(_layer_body, n),
        grid=(NBLK,),
        in_specs=agg_specs + [
            pl.BlockSpec((BN, 1), lambda i: (i, 0)),
            pl.BlockSpec((BN, D), lambda i: (i, 0)),
        ] + w1_specs + [
            pl.BlockSpec((D, 32), lambda i: (0, 0)),
            pl.BlockSpec((1, 32), lambda i: (0, 0)),
            pl.BlockSpec((32, 1), lambda i: (0, 0)),
            pl.BlockSpec((BN, 1), lambda i: (i, 0)),
        ],
        out_specs=[
            pl.BlockSpec((BN, 32), lambda i: (i, 0)),
            pl.BlockSpec((BN, 1), lambda i: (i, 0)),
        ],
        out_shape=[
            jax.ShapeDtypeStruct((N_PAD, 32), jnp.float32),
            jax.ShapeDtypeStruct((N_PAD, 1), jnp.float32),
        ],
    )(*aggs, bm, xin, *w1s, w2, b, pn, m)


def _topk_body(k, y_ref, crit_o):
    u = _u32(y_ref[...])

    def bit(i, t):
        cand = t | (jnp.uint32(1) << jnp.uint32(31 - i))
        cnt = jnp.sum((u >= cand).astype(jnp.int32))
        return jnp.where(cnt >= k, cand, t)
    t = lax.fori_loop(0, 32, bit, jnp.uint32(0))

    cnt_gt = jnp.sum((u > t).astype(jnp.int32))
    r = k - cnt_gt
    tie = u == t
    idx = (lax.broadcasted_iota(jnp.int32, (392, 128), 0) * 128
           + lax.broadcasted_iota(jnp.int32, (392, 128), 1))

    def jbit(i, m):
        cand = m | (1 << (16 - i))
        f = jnp.sum((tie & (idx < cand)).astype(jnp.int32))
        return jnp.where(f < r, cand, m)
    m = lax.fori_loop(0, 17, jbit, jnp.int32(0))
    n_r = jnp.where(r > 0, m + 1, 0)

    lane = lax.broadcasted_iota(jnp.int32, (1, 128), 1)
    t_i = lax.bitcast_convert_type(t, jnp.int32)
    crit_o[...] = jnp.where(lane == 0, t_i, jnp.where(lane == 1, n_r, 0))


def _tc_topk(y2d, k):
    return pl.pallas_call(
        functools.partial(_topk_body, k),
        out_shape=jax.ShapeDtypeStruct((1, 128), jnp.int32),
    )(y2d)


def _gate_body(y_ref, h_ref, a0_ref, b0_ref, crit_ref, T_o, H_o, bm_o, m_o):
    pid = pl.program_id(0)
    t_u = lax.bitcast_convert_type(crit_ref[0, 0], jnp.uint32)
    n_r = crit_ref[0, 1]
    y = y_ref[...]
    u = _u32(y)
    rows = pid * BN + lax.broadcasted_iota(jnp.int32, (BN, 1), 0)
    sel = (u > t_u) | ((u == t_u) & (rows < n_r))
    mf = sel.astype(jnp.float32)
    g = jnp.tanh(y) * mf
    Hrow = h_ref[...] * g
    H_o[...] = Hrow
    T_o[...] = Hrow * (a0_ref[...] * mf)
    bm_o[...] = b0_ref[...] * mf
    m_o[...] = mf


def _tc_gate(y, h, a0, b0, crit):
    return pl.pallas_call(
        _gate_body,
        grid=(NBLK,),
        in_specs=[
            pl.BlockSpec((BN, 1), lambda i: (i, 0)),
            pl.BlockSpec((BN, 32), lambda i: (i, 0)),
            pl.BlockSpec((BN, 1), lambda i: (i, 0)),
            pl.BlockSpec((BN, 1), lambda i: (i, 0)),
            pl.BlockSpec((1, 128), lambda i: (0, 0)),
        ],
        out_specs=[
            pl.BlockSpec((BN, 32), lambda i: (i, 0)),
            pl.BlockSpec((BN, 32), lambda i: (i, 0)),
            pl.BlockSpec((BN, 1), lambda i: (i, 0)),
            pl.BlockSpec((BN, 1), lambda i: (i, 0)),
        ],
        out_shape=[
            jax.ShapeDtypeStruct((N_PAD, 32), jnp.float32),
            jax.ShapeDtypeStruct((N_PAD, 32), jnp.float32),
            jax.ShapeDtypeStruct((N_PAD, 1), jnp.float32),
            jax.ShapeDtypeStruct((N_PAD, 1), jnp.float32),
        ],
    )(y, h, a0, b0, crit)


def _final_body(nparts, *refs):
    aggs = refs[:nparts]
    bm_ref, H_ref = refs[nparts:nparts + 2]
    w1s = refs[nparts + 2:2 * nparts + 2]
    m_ref, w2_ref, b_ref, wd_ref, bd_ref, out_o, acc = refs[2 * nparts + 2:]
    pid = pl.program_id(0)

    @pl.when(pid == 0)
    def _():
        acc[...] = jnp.zeros((1, 32), jnp.float32)

    z = _layer_z(aggs, w1s, bm_ref, H_ref, w2_ref, b_ref)
    h2 = jnp.maximum(z, jnp.float32(0)) * m_ref[...]
    acc[...] += jnp.sum(h2, axis=0, keepdims=True)

    @pl.when(pid == NBLK - 1)
    def _():
        pooled = acc[...] / jnp.float32(K2)
        logits = (jnp.dot(pooled, wd_ref[...],
                          preferred_element_type=jnp.float32) + bd_ref[...])
        mx = jnp.max(logits, axis=-1, keepdims=True)
        e = jnp.exp(logits - mx)
        out_o[...] = e / jnp.sum(e, axis=-1, keepdims=True)


def _tc_final(aggs, bm, H, w1s, m, w2, b, wd, bd):
    n = len(aggs)
    agg_specs = [pl.BlockSpec((2, BN, a.shape[2]), lambda i: (0, i, 0))
                 for a in aggs]
    w1_specs = [pl.BlockSpec(w.shape, lambda i: (0, 0)) for w in w1s]
    return pl.pallas_call(
        functools.partial(_final_body, n),
        grid=(NBLK,),
        in_specs=agg_specs + [
            pl.BlockSpec((BN, 1), lambda i: (i, 0)),
            pl.BlockSpec((BN, 32), lambda i: (i, 0)),
        ] + w1_specs + [
            pl.BlockSpec((BN, 1), lambda i: (i, 0)),
            pl.BlockSpec((32, 32), lambda i: (0, 0)),
            pl.BlockSpec((1, 32), lambda i: (0, 0)),
            pl.BlockSpec((32, 3), lambda i: (0, 0)),
            pl.BlockSpec((1, 3), lambda i: (0, 0)),
        ],
        out_specs=pl.BlockSpec((1, 3), lambda i: (0, 0)),
        out_shape=jax.ShapeDtypeStruct((1, 3), jnp.float32),
        scratch_shapes=[pltpu.VMEM((1, 32), jnp.float32)],
    )(*aggs, bm, H, *w1s, m, w2, b, wd, bd)


def kernel(x, W1a, W2a, ba, p, W1b, W2b, bb, W1c, W2c, bc, Wd, bd,
           edge_index, i):
    f32 = jnp.float32
    src = edge_index[0]
    dst = edge_index[1]
    pad = jnp.full((E_PAD - E,), TRASH, jnp.int32)
    srcr = jnp.concatenate([src, pad]).reshape(NW, GRP * NGRP, CHUNK)
    dstr = jnp.concatenate([dst, pad]).reshape(NW, GRP * NGRP, CHUNK)

    x4 = jnp.zeros((N_PAD, 4), f32).at[:N, :3].set(x)
    W1a4 = jnp.zeros((4, 32), f32).at[:3].set(W1a)
    W2a4 = jnp.zeros((4, 32), f32).at[:3].set(W2a)
    pn = (p / jnp.linalg.norm(p)).reshape(32, 1)
    valid = (jnp.arange(N_PAD) < N).astype(f32).reshape(N_PAD, 1)
    zeros1 = jnp.zeros((N_PAD,), f32)
    zeros4 = jnp.zeros((N_PAD, 4), f32)
    zeros32 = jnp.zeros((N_PAD, 32), f32)

    s0, s1, d0, d1 = _sc_deg(srcr, dstr, zeros1)
    a0, b0, T0 = _tc_deg(s0, s1, d0, d1, x4)
    agg0 = _sc_agg(srcr, dstr, T0, zeros4)
    h, y1 = _tc_layer([agg0], b0, x4, [W1a4], W2a4, ba.reshape(1, 32), pn,
                      valid)
    crit1 = _tc_topk(y1.reshape(392, 128), K1)
    T1, H1, bm1, m1 = _tc_gate(y1, h, a0, b0, crit1)
    agg1 = [_sc_agg(srcr, dstr, T1, zeros32)]
    h1, y2 = _tc_layer(agg1, bm1, H1, [W1b], W2b,
                       bb.reshape(1, 32), pn, m1)
    crit2 = _tc_topk(y2.reshape(392, 128), K2)
    T2, H2, bm2, m2 = _tc_gate(y2, h1, a0, b0, crit2)
    agg2 = [_sc_agg(srcr, dstr, T2, zeros32)]
    return _tc_final(agg2, bm2, H2, [W1c], m2, W2c,
                     bc.reshape(1, 32), Wd, bd.reshape(1, 3))


# speedup vs baseline: 58.0547x; 1.1035x over previous
import functools

import jax
import jax.numpy as jnp
from jax import lax
from jax.experimental import pallas as pl
from jax.experimental.pallas import tpu as pltpu
from jax.experimental.pallas import tpu_sc as plsc

N = 50000
E = 1600000
N_PAD = 50176
NW = 32
CHUNK = 256
GRP = 4
NGRP = 49
CHUNK_W = 1024
GRP_W = 7
NGRP_W = 7
E_PER_TILE = CHUNK * GRP * NGRP
E_PAD = NW * E_PER_TILE
ROWS_PER_TILE = N_PAD // 16
CP = 112
TRASH = N
K1 = 25000
K2 = 12500
BN = 1024
NBLK = N_PAD // BN

@functools.cache
def _mesh():
    return plsc.VectorSubcoreMesh(core_axis_name="c", subcore_axis_name="s")


def _deg_body(srcr, dstr, zeros1, outS0, outS1, outD0, outD1,
              degS, degD, ones_v, idx_v, buf_v, dsem):
    c = lax.axis_index("c")
    s = lax.axis_index("s")
    w = c * 16 + s

    def setones(j, _):
        ones_v[pl.ds(j * 16, 16)] = jnp.ones((16,), jnp.float32)
        return 0
    lax.fori_loop(0, CHUNK_W // 16, setones, 0)

    row0 = s * ROWS_PER_TILE
    pltpu.sync_copy(zeros1.at[pl.ds(0, CP)], buf_v)

    def zchunk(t, _):
        pltpu.sync_copy(buf_v, degS.at[pl.ds(row0 + t * CP, CP)])
        pltpu.sync_copy(buf_v, degD.at[pl.ds(row0 + t * CP, CP)])
        return 0
    lax.fori_loop(0, ROWS_PER_TILE // CP, zchunk, 0)
    plsc.subcore_barrier()

    def grp(g, _):
        pltpu.sync_copy(srcr.at[w, pl.ds(g * GRP_W, GRP_W)], idx_v)
        descs = [pltpu.async_copy(ones_v, degS.at[idx_v.at[j]], dsem,
                                  add=True) for j in range(GRP_W)]
        for dsc in descs:
            dsc.wait()
        pltpu.sync_copy(dstr.at[w, pl.ds(g * GRP_W, GRP_W)], idx_v)
        descs = [pltpu.async_copy(ones_v, degD.at[idx_v.at[j]], dsem,
                                  add=True) for j in range(GRP_W)]
        for dsc in descs:
            dsc.wait()
        return 0
    lax.fori_loop(0, NGRP_W, grp, 0)

    plsc.subcore_barrier()

    def out1(acc, dstref):
        def cp(t, _):
            sl = pl.ds(row0 + t * CP, CP)
            pltpu.sync_copy(acc.at[sl], buf_v)
            pltpu.sync_copy(buf_v, dstref.at[sl])
            return 0
        lax.fori_loop(0, ROWS_PER_TILE // CP, cp, 0)

    @pl.when(c == 0)
    def _():
        out1(degS, outS0)
        out1(degD, outD0)

    @pl.when(c == 1)
    def _():
        out1(degS, outS1)
        out1(degD, outD1)


@functools.cache
def _sc_deg_fn():
    return pl.kernel(
        _deg_body,
        out_type=[jax.ShapeDtypeStruct((N_PAD,), jnp.float32)] * 4,
        mesh=_mesh(),
        compiler_params=pltpu.CompilerParams(use_tc_tiling_on_sc=False),
        scratch_types=[
            pltpu.VMEM_SHARED((N_PAD,), jnp.float32),
            pltpu.VMEM_SHARED((N_PAD,), jnp.float32),
            pltpu.VMEM((CHUNK_W,), jnp.float32),
            pltpu.VMEM((GRP_W, CHUNK_W), jnp.int32),
            pltpu.VMEM((CP,), jnp.float32),
            pltpu.SemaphoreType.DMA,
        ],
    )


def _sc_deg(srcr, dstr, zeros1):
    return _sc_deg_fn()(srcr, dstr, zeros1)


def _agg_body(D, grpn, ngrp, srcr, dstr, tab, zerosD, out, acc, idxs_v,
              idxd_v, r0_v, r1_v, buf_v, gs0, gs1, ss0, ss1):
    c = lax.axis_index("c")
    s = lax.axis_index("s")
    w = c * 16 + s

    row0 = s * ROWS_PER_TILE
    pltpu.sync_copy(zerosD.at[pl.ds(0, CP)], buf_v)

    def zchunk(t, _):
        pltpu.sync_copy(buf_v, acc.at[pl.ds(row0 + t * CP, CP)])
        return 0
    lax.fori_loop(0, ROWS_PER_TILE // CP, zchunk, 0)
    plsc.subcore_barrier()

    rows = (r0_v, r1_v)
    gsem = (gs0, gs1)
    ssem = (ss0, ss1)

    def grp(g, _):
        pltpu.sync_copy(srcr.at[w, pl.ds(g * grpn, grpn)], idxs_v)
        pltpu.sync_copy(dstr.at[w, pl.ds(g * grpn, grpn)], idxd_v)
        gd = [None, None]
        sd = [None, None]
        gd[0] = pltpu.async_copy(tab.at[idxs_v.at[0]], rows[0], gsem[0])
        for j in range(grpn):
            b = j % 2
            gd[b].wait()
            if j >= 1:
                sd[1 - b].wait()
            sd[b] = pltpu.async_copy(rows[b], acc.at[idxd_v.at[j]],
                                     ssem[b], add=True)
            if j + 1 < grpn:
                gd[1 - b] = pltpu.async_copy(tab.at[idxs_v.at[j + 1]],
                                             rows[1 - b], gsem[1 - b])
        sd[(grpn - 1) % 2].wait()
        return 0
    lax.fori_loop(0, ngrp, grp, 0)

    plsc.subcore_barrier()

    def cp(t, _):
        sl = pl.ds(row0 + t * CP, CP)
        pltpu.sync_copy(acc.at[sl], buf_v)
        pltpu.sync_copy(buf_v, out.at[c, sl])
        return 0
    lax.fori_loop(0, ROWS_PER_TILE // CP, cp, 0)


@functools.cache
def _sc_agg_fn(D, chk, grpn, ngrp):
    return pl.kernel(
        functools.partial(_agg_body, D, grpn, ngrp),
        out_type=jax.ShapeDtypeStruct((2, N_PAD, D), jnp.float32),
        mesh=_mesh(),
        compiler_params=pltpu.CompilerParams(use_tc_tiling_on_sc=False),
        scratch_types=[
            pltpu.VMEM_SHARED((N_PAD, D), jnp.float32),
            pltpu.VMEM((grpn, chk), jnp.int32),
            pltpu.VMEM((grpn, chk), jnp.int32),
            pltpu.VMEM((chk, D), jnp.float32),
            pltpu.VMEM((chk, D), jnp.float32),
            pltpu.VMEM((CP, D), jnp.float32),
            pltpu.SemaphoreType.DMA,
            pltpu.SemaphoreType.DMA,
            pltpu.SemaphoreType.DMA,
            pltpu.SemaphoreType.DMA,
        ],
    )


def _sc_agg(srcr, dstr, tab, zerosD):
    chk = srcr.shape[2]
    grpn = GRP_W if chk == CHUNK_W else GRP
    ngrp = NGRP_W if chk == CHUNK_W else NGRP
    return _sc_agg_fn(tab.shape[1], chk, grpn, ngrp)(srcr, dstr, tab, zerosD)


def _u32(y):
    u = lax.bitcast_convert_type(y, jnp.uint32)
    return jnp.where((u >> jnp.uint32(31)) > jnp.uint32(0),
                     ~u, u | jnp.uint32(0x80000000))


def _deg_tc_body(s0_ref, s1_ref, d0_ref, d1_ref, x4_ref, a0_o, b0_o, t0_o):
    ds = s0_ref[...] + s1_ref[...]
    dd = d0_ref[...] + d1_ref[...]
    a0 = jnp.where(ds > 0, lax.rsqrt(jnp.maximum(ds, 1e-12)),
                   jnp.float32(0))
    b0 = jnp.where(dd > 0, lax.rsqrt(jnp.maximum(dd, 1e-12)),
                   jnp.float32(0))
    a0_o[...] = a0
    b0_o[...] = b0
    t0_o[...] = x4_ref[...] * a0


def _tc_deg(s0, s1, d0, d1, x4):
    return pl.pallas_call(
        _deg_tc_body,
        grid=(NBLK,),
        in_specs=[
            pl.BlockSpec((BN, 1), lambda i: (i, 0)),
            pl.BlockSpec((BN, 1), lambda i: (i, 0)),
            pl.BlockSpec((BN, 1), lambda i: (i, 0)),
            pl.BlockSpec((BN, 1), lambda i: (i, 0)),
            pl.BlockSpec((BN, 4), lambda i: (i, 0)),
        ],
        out_specs=[
            pl.BlockSpec((BN, 1), lambda i: (i, 0)),
            pl.BlockSpec((BN, 1), lambda i: (i, 0)),
            pl.BlockSpec((BN, 4), lambda i: (i, 0)),
        ],
        out_shape=[
            jax.ShapeDtypeStruct((N_PAD, 1), jnp.float32),
            jax.ShapeDtypeStruct((N_PAD, 1), jnp.float32),
            jax.ShapeDtypeStruct((N_PAD, 4), jnp.float32),
        ],
    )(s0.reshape(N_PAD, 1), s1.reshape(N_PAD, 1),
      d0.reshape(N_PAD, 1), d1.reshape(N_PAD, 1), x4)


def _layer_z(aggs, w1s, bm_ref, xin_ref, w2_ref, b_ref):
    z = (jnp.dot(xin_ref[...], w2_ref[...],
                 preferred_element_type=jnp.float32) + b_ref[...])
    for ap, w1 in zip(aggs, w1s):
        agg = ap[0] + ap[1]
        z = z + jnp.dot(agg * bm_ref[...], w1[...],
                        preferred_element_type=jnp.float32)
    return z


def _layer_body(nparts, *refs):
    aggs = refs[:nparts]
    bm_ref, xin_ref = refs[nparts:nparts + 2]
    w1s = refs[nparts + 2:2 * nparts + 2]
    w2_ref, b_ref, pn_ref, m_ref, h_o, y_o = refs[2 * nparts + 2:]
    h = jnp.maximum(_layer_z(aggs, w1s, bm_ref, xin_ref, w2_ref, b_ref),
                    jnp.float32(0))
    h_o[...] = h
    y = jnp.dot(h, pn_ref[...], preferred_element_type=jnp.float32)
    y_o[...] = jnp.where(m_ref[...] > 0, y, jnp.float32(-jnp.inf))


def _tc_layer(aggs, bm, xin, w1s, w2, b, pn, m):
    D = xin.shape[1]
    n = len(aggs)
    agg_specs = [pl.BlockSpec((2, BN, a.shape[2]), lambda i: (0, i, 0))
                 for a in aggs]
    w1_specs = [pl.BlockSpec(w.shape, lambda i: (0, 0)) for w in w1s]
    return pl.pallas_call(
        functools.partial(_layer_body, n),
        grid=(NBLK,),
        in_specs=agg_specs + [
            pl.BlockSpec((BN, 1), lambda i: (i, 0)),
            pl.BlockSpec((BN, D), lambda i: (i, 0)),
        ] + w1_specs + [
            pl.BlockSpec((D, 32), lambda i: (0, 0)),
            pl.BlockSpec((1, 32), lambda i: (0, 0)),
            pl.BlockSpec((32, 1), lambda i: (0, 0)),
            pl.BlockSpec((BN, 1), lambda i: (i, 0)),
        ],
        out_specs=[
            pl.BlockSpec((BN, 32), lambda i: (i, 0)),
            pl.BlockSpec((BN, 1), lambda i: (i, 0)),
        ],
        out_shape=[
            jax.ShapeDtypeStruct((N_PAD, 32), jnp.float32),
            jax.ShapeDtypeStruct((N_PAD, 1), jnp.float32),
        ],
    )(*aggs, bm, xin, *w1s, w2, b, pn, m)


def _topk_body(k, y_ref, crit_o):
    u = _u32(y_ref[...])

    def bit(i, t):
        cand = t | (jnp.uint32(1) << jnp.uint32(31 - i))
        cnt = jnp.sum((u >= cand).astype(jnp.int32))
        return jnp.where(cnt >= k, cand, t)
    t = lax.fori_loop(0, 32, bit, jnp.uint32(0))

    cnt_gt = jnp.sum((u > t).astype(jnp.int32))
    r = k - cnt_gt
    tie = u == t
    idx = (lax.broadcasted_iota(jnp.int32, (392, 128), 0) * 128
           + lax.broadcasted_iota(jnp.int32, (392, 128), 1))

    def jbit(i, m):
        cand = m | (1 << (16 - i))
        f = jnp.sum((tie & (idx < cand)).astype(jnp.int32))
        return jnp.where(f < r, cand, m)
    m = lax.fori_loop(0, 17, jbit, jnp.int32(0))
    n_r = jnp.where(r > 0, m + 1, 0)

    lane = lax.broadcasted_iota(jnp.int32, (1, 128), 1)
    t_i = lax.bitcast_convert_type(t, jnp.int32)
    crit_o[...] = jnp.where(lane == 0, t_i, jnp.where(lane == 1, n_r, 0))


def _tc_topk(y2d, k):
    return pl.pallas_call(
        functools.partial(_topk_body, k),
        out_shape=jax.ShapeDtypeStruct((1, 128), jnp.int32),
    )(y2d)


def _gate_body(y_ref, h_ref, a0_ref, b0_ref, crit_ref, T_o, H_o, bm_o, m_o):
    pid = pl.program_id(0)
    t_u = lax.bitcast_convert_type(crit_ref[0, 0], jnp.uint32)
    n_r = crit_ref[0, 1]
    y = y_ref[...]
    u = _u32(y)
    rows = pid * BN + lax.broadcasted_iota(jnp.int32, (BN, 1), 0)
    sel = (u > t_u) | ((u == t_u) & (rows < n_r))
    mf = sel.astype(jnp.float32)
    g = jnp.tanh(y) * mf
    Hrow = h_ref[...] * g
    H_o[...] = Hrow
    T_o[...] = Hrow * (a0_ref[...] * mf)
    bm_o[...] = b0_ref[...] * mf
    m_o[...] = mf


def _tc_gate(y, h, a0, b0, crit):
    return pl.pallas_call(
        _gate_body,
        grid=(NBLK,),
        in_specs=[
            pl.BlockSpec((BN, 1), lambda i: (i, 0)),
            pl.BlockSpec((BN, 32), lambda i: (i, 0)),
            pl.BlockSpec((BN, 1), lambda i: (i, 0)),
            pl.BlockSpec((BN, 1), lambda i: (i, 0)),
            pl.BlockSpec((1, 128), lambda i: (0, 0)),
        ],
        out_specs=[
            pl.BlockSpec((BN, 32), lambda i: (i, 0)),
            pl.BlockSpec((BN, 32), lambda i: (i, 0)),
            pl.BlockSpec((BN, 1), lambda i: (i, 0)),
            pl.BlockSpec((BN, 1), lambda i: (i, 0)),
        ],
        out_shape=[
            jax.ShapeDtypeStruct((N_PAD, 32), jnp.float32),
            jax.ShapeDtypeStruct((N_PAD, 32), jnp.float32),
            jax.ShapeDtypeStruct((N_PAD, 1), jnp.float32),
            jax.ShapeDtypeStruct((N_PAD, 1), jnp.float32),
        ],
    )(y, h, a0, b0, crit)


def _final_body(nparts, *refs):
    aggs = refs[:nparts]
    bm_ref, H_ref = refs[nparts:nparts + 2]
    w1s = refs[nparts + 2:2 * nparts + 2]
    m_ref, w2_ref, b_ref, wd_ref, bd_ref, out_o, acc = refs[2 * nparts + 2:]
    pid = pl.program_id(0)

    @pl.when(pid == 0)
    def _():
        acc[...] = jnp.zeros((1, 32), jnp.float32)

    z = _layer_z(aggs, w1s, bm_ref, H_ref, w2_ref, b_ref)
    h2 = jnp.maximum(z, jnp.float32(0)) * m_ref[...]
    acc[...] += jnp.sum(h2, axis=0, keepdims=True)

    @pl.when(pid == NBLK - 1)
    def _():
        pooled = acc[...] / jnp.float32(K2)
        logits = (jnp.dot(pooled, wd_ref[...],
                          preferred_element_type=jnp.float32) + bd_ref[...])
        mx = jnp.max(logits, axis=-1, keepdims=True)
        e = jnp.exp(logits - mx)
        out_o[...] = e / jnp.sum(e, axis=-1, keepdims=True)


def _tc_final(aggs, bm, H, w1s, m, w2, b, wd, bd):
    n = len(aggs)
    agg_specs = [pl.BlockSpec((2, BN, a.shape[2]), lambda i: (0, i, 0))
                 for a in aggs]
    w1_specs = [pl.BlockSpec(w.shape, lambda i: (0, 0)) for w in w1s]
    return pl.pallas_call(
        functools.partial(_final_body, n),
        grid=(NBLK,),
        in_specs=agg_specs + [
            pl.BlockSpec((BN, 1), lambda i: (i, 0)),
            pl.BlockSpec((BN, 32), lambda i: (i, 0)),
        ] + w1_specs + [
            pl.BlockSpec((BN, 1), lambda i: (i, 0)),
            pl.BlockSpec((32, 32), lambda i: (0, 0)),
            pl.BlockSpec((1, 32), lambda i: (0, 0)),
            pl.BlockSpec((32, 3), lambda i: (0, 0)),
            pl.BlockSpec((1, 3), lambda i: (0, 0)),
        ],
        out_specs=pl.BlockSpec((1, 3), lambda i: (0, 0)),
        out_shape=jax.ShapeDtypeStruct((1, 3), jnp.float32),
        scratch_shapes=[pltpu.VMEM((1, 32), jnp.float32)],
    )(*aggs, bm, H, *w1s, m, w2, b, wd, bd)


def kernel(x, W1a, W2a, ba, p, W1b, W2b, bb, W1c, W2c, bc, Wd, bd,
           edge_index, i):
    f32 = jnp.float32
    src = edge_index[0]
    dst = edge_index[1]
    pad = jnp.full((E_PAD - E,), TRASH, jnp.int32)
    srcp = jnp.concatenate([src, pad])
    dstp = jnp.concatenate([dst, pad])
    srcr = srcp.reshape(NW, GRP * NGRP, CHUNK)
    dstr = dstp.reshape(NW, GRP * NGRP, CHUNK)
    srcw = srcp.reshape(NW, GRP_W * NGRP_W, CHUNK_W)
    dstw = dstp.reshape(NW, GRP_W * NGRP_W, CHUNK_W)

    x4 = jnp.zeros((N_PAD, 4), f32).at[:N, :3].set(x)
    W1a4 = jnp.zeros((4, 32), f32).at[:3].set(W1a)
    W2a4 = jnp.zeros((4, 32), f32).at[:3].set(W2a)
    pn = (p / jnp.linalg.norm(p)).reshape(32, 1)
    valid = (jnp.arange(N_PAD) < N).astype(f32).reshape(N_PAD, 1)
    zeros1 = jnp.zeros((N_PAD,), f32)
    zeros4 = jnp.zeros((N_PAD, 4), f32)
    zeros32 = jnp.zeros((N_PAD, 32), f32)

    s0, s1, d0, d1 = _sc_deg(srcw, dstw, zeros1)
    a0, b0, T0 = _tc_deg(s0, s1, d0, d1, x4)
    agg0 = _sc_agg(srcw, dstw, T0, zeros4)
    h, y1 = _tc_layer([agg0], b0, x4, [W1a4], W2a4, ba.reshape(1, 32), pn,
                      valid)
    crit1 = _tc_topk(y1.reshape(392, 128), K1)
    T1, H1, bm1, m1 = _tc_gate(y1, h, a0, b0, crit1)
    agg1 = [_sc_agg(srcr, dstr, T1, zeros32)]
    h1, y2 = _tc_layer(agg1, bm1, H1, [W1b], W2b,
                       bb.reshape(1, 32), pn, m1)
    crit2 = _tc_topk(y2.reshape(392, 128), K2)
    T2, H2, bm2, m2 = _tc_gate(y2, h1, a0, b0, crit2)
    agg2 = [_sc_agg(srcr, dstr, T2, zeros32)]
    return _tc_final(agg2, bm2, H2, [W1c], m2, W2c,
                     bc.reshape(1, 32), Wd, bd.reshape(1, 3))


# speedup vs baseline: 58.4664x; 1.0071x over previous
import functools

import jax
import jax.numpy as jnp
from jax import lax
from jax.experimental import pallas as pl
from jax.experimental.pallas import tpu as pltpu
from jax.experimental.pallas import tpu_sc as plsc

N = 50000
E = 1600000
N_PAD = 50176
NW = 32
CHUNK = 256
GRP = 4
NGRP = 49
CHUNK_W = 1024
GRP_W = 7
NGRP_W = 7
E_PER_TILE = CHUNK * GRP * NGRP
E_PAD = NW * E_PER_TILE
ROWS_PER_TILE = N_PAD // 16
CP = 112
TRASH = N
K1 = 25000
K2 = 12500
BN = 1024
NBLK = N_PAD // BN

@functools.cache
def _mesh():
    return plsc.VectorSubcoreMesh(core_axis_name="c", subcore_axis_name="s")


def _deg_body(srcr, dstr, zeros1, outS0, outS1, outD0, outD1,
              degS, degD, ones_v, idx_v, buf_v, dsem):
    c = lax.axis_index("c")
    s = lax.axis_index("s")
    w = c * 16 + s

    def setones(j, _):
        ones_v[pl.ds(j * 16, 16)] = jnp.ones((16,), jnp.float32)
        return 0
    lax.fori_loop(0, CHUNK_W // 16, setones, 0)

    row0 = s * ROWS_PER_TILE
    pltpu.sync_copy(zeros1.at[pl.ds(0, CP)], buf_v)

    def zchunk(t, _):
        pltpu.sync_copy(buf_v, degS.at[pl.ds(row0 + t * CP, CP)])
        pltpu.sync_copy(buf_v, degD.at[pl.ds(row0 + t * CP, CP)])
        return 0
    lax.fori_loop(0, ROWS_PER_TILE // CP, zchunk, 0)
    plsc.subcore_barrier()

    def grp(g, _):
        pltpu.sync_copy(srcr.at[w, pl.ds(g * GRP_W, GRP_W)], idx_v)
        descs = [pltpu.async_copy(ones_v, degS.at[idx_v.at[j]], dsem,
                                  add=True) for j in range(GRP_W)]
        for dsc in descs:
            dsc.wait()
        pltpu.sync_copy(dstr.at[w, pl.ds(g * GRP_W, GRP_W)], idx_v)
        descs = [pltpu.async_copy(ones_v, degD.at[idx_v.at[j]], dsem,
                                  add=True) for j in range(GRP_W)]
        for dsc in descs:
            dsc.wait()
        return 0
    lax.fori_loop(0, NGRP_W, grp, 0)

    plsc.subcore_barrier()

    def out1(acc, dstref):
        def cp(t, _):
            sl = pl.ds(row0 + t * CP, CP)
            pltpu.sync_copy(acc.at[sl], buf_v)
            pltpu.sync_copy(buf_v, dstref.at[sl])
            return 0
        lax.fori_loop(0, ROWS_PER_TILE // CP, cp, 0)

    @pl.when(c == 0)
    def _():
        out1(degS, outS0)
        out1(degD, outD0)

    @pl.when(c == 1)
    def _():
        out1(degS, outS1)
        out1(degD, outD1)


@functools.cache
def _sc_deg_fn():
    return pl.kernel(
        _deg_body,
        out_type=[jax.ShapeDtypeStruct((N_PAD,), jnp.float32)] * 4,
        mesh=_mesh(),
        compiler_params=pltpu.CompilerParams(use_tc_tiling_on_sc=False),
        scratch_types=[
            pltpu.VMEM_SHARED((N_PAD,), jnp.float32),
            pltpu.VMEM_SHARED((N_PAD,), jnp.float32),
            pltpu.VMEM((CHUNK_W,), jnp.float32),
            pltpu.VMEM((GRP_W, CHUNK_W), jnp.int32),
            pltpu.VMEM((CP,), jnp.float32),
            pltpu.SemaphoreType.DMA,
        ],
    )


def _sc_deg(srcr, dstr, zeros1):
    return _sc_deg_fn()(srcr, dstr, zeros1)


def _agg_body(D, grpn, ngrp, srcr, dstr, tab, zerosD, out, acc, idxs_v,
              idxd_v, r0_v, r1_v, buf_v, gs0, gs1, ss0, ss1):
    c = lax.axis_index("c")
    s = lax.axis_index("s")
    w = c * 16 + s

    row0 = s * ROWS_PER_TILE
    pltpu.sync_copy(zerosD.at[pl.ds(0, CP)], buf_v)

    def zchunk(t, _):
        pltpu.sync_copy(buf_v, acc.at[pl.ds(row0 + t * CP, CP)])
        return 0
    lax.fori_loop(0, ROWS_PER_TILE // CP, zchunk, 0)
    plsc.subcore_barrier()

    rows = (r0_v, r1_v)
    gsem = (gs0, gs1)
    ssem = (ss0, ss1)

    def grp(g, _):
        pltpu.sync_copy(srcr.at[w, pl.ds(g * grpn, grpn)], idxs_v)
        pltpu.sync_copy(dstr.at[w, pl.ds(g * grpn, grpn)], idxd_v)
        gd = [None, None]
        sd = [None, None]
        gd[0] = pltpu.async_copy(tab.at[idxs_v.at[0]], rows[0], gsem[0])
        for j in range(grpn):
            b = j % 2
            gd[b].wait()
            if j >= 1:
                sd[1 - b].wait()
            sd[b] = pltpu.async_copy(rows[b], acc.at[idxd_v.at[j]],
                                     ssem[b], add=True)
            if j + 1 < grpn:
                gd[1 - b] = pltpu.async_copy(tab.at[idxs_v.at[j + 1]],
                                             rows[1 - b], gsem[1 - b])
        sd[(grpn - 1) % 2].wait()
        return 0
    lax.fori_loop(0, ngrp, grp, 0)

    plsc.subcore_barrier()

    def cp(t, _):
        sl = pl.ds(row0 + t * CP, CP)
        pltpu.sync_copy(acc.at[sl], buf_v)
        pltpu.sync_copy(buf_v, out.at[c, sl])
        return 0
    lax.fori_loop(0, ROWS_PER_TILE // CP, cp, 0)


@functools.cache
def _sc_agg_fn(D, chk, grpn, ngrp):
    return pl.kernel(
        functools.partial(_agg_body, D, grpn, ngrp),
        out_type=jax.ShapeDtypeStruct((2, N_PAD, D), jnp.float32),
        mesh=_mesh(),
        compiler_params=pltpu.CompilerParams(use_tc_tiling_on_sc=False),
        scratch_types=[
            pltpu.VMEM_SHARED((N_PAD, D), jnp.float32),
            pltpu.VMEM((grpn, chk), jnp.int32),
            pltpu.VMEM((grpn, chk), jnp.int32),
            pltpu.VMEM((chk, D), jnp.float32),
            pltpu.VMEM((chk, D), jnp.float32),
            pltpu.VMEM((CP, D), jnp.float32),
            pltpu.SemaphoreType.DMA,
            pltpu.SemaphoreType.DMA,
            pltpu.SemaphoreType.DMA,
            pltpu.SemaphoreType.DMA,
        ],
    )


def _sc_agg(srcr, dstr, tab, zerosD):
    chk = srcr.shape[2]
    grpn = GRP_W if chk == CHUNK_W else GRP
    ngrp = NGRP_W if chk == CHUNK_W else NGRP
    return _sc_agg_fn(tab.shape[1], chk, grpn, ngrp)(srcr, dstr, tab, zerosD)


def _u32(y):
    u = lax.bitcast_convert_type(y, jnp.uint32)
    return jnp.where((u >> jnp.uint32(31)) > jnp.uint32(0),
                     ~u, u | jnp.uint32(0x80000000))


def _deg_tc_body(s0_ref, s1_ref, d0_ref, d1_ref, x4_ref, a0_o, b0_o, t0_o):
    ds = s0_ref[...] + s1_ref[...]
    dd = d0_ref[...] + d1_ref[...]
    a0 = jnp.where(ds > 0, lax.rsqrt(jnp.maximum(ds, 1e-12)),
                   jnp.float32(0))
    b0 = jnp.where(dd > 0, lax.rsqrt(jnp.maximum(dd, 1e-12)),
                   jnp.float32(0))
    a0_o[...] = a0
    b0_o[...] = b0
    t0_o[...] = x4_ref[...] * a0


def _tc_deg(s0, s1, d0, d1, x4):
    return pl.pallas_call(
        _deg_tc_body,
        grid=(NBLK,),
        in_specs=[
            pl.BlockSpec((BN, 1), lambda i: (i, 0)),
            pl.BlockSpec((BN, 1), lambda i: (i, 0)),
            pl.BlockSpec((BN, 1), lambda i: (i, 0)),
            pl.BlockSpec((BN, 1), lambda i: (i, 0)),
            pl.BlockSpec((BN, 4), lambda i: (i, 0)),
        ],
        out_specs=[
            pl.BlockSpec((BN, 1), lambda i: (i, 0)),
            pl.BlockSpec((BN, 1), lambda i: (i, 0)),
            pl.BlockSpec((BN, 4), lambda i: (i, 0)),
        ],
        out_shape=[
            jax.ShapeDtypeStruct((N_PAD, 1), jnp.float32),
            jax.ShapeDtypeStruct((N_PAD, 1), jnp.float32),
            jax.ShapeDtypeStruct((N_PAD, 4), jnp.float32),
        ],
    )(s0.reshape(N_PAD, 1), s1.reshape(N_PAD, 1),
      d0.reshape(N_PAD, 1), d1.reshape(N_PAD, 1), x4)


def _layer_z(aggs, w1s, bm_ref, xin_ref, w2_ref, b_ref):
    z = (jnp.dot(xin_ref[...], w2_ref[...],
                 preferred_element_type=jnp.float32) + b_ref[...])
    for ap, w1 in zip(aggs, w1s):
        agg = ap[0] + ap[1]
        z = z + jnp.dot(agg * bm_ref[...], w1[...],
                        preferred_element_type=jnp.float32)
    return z


def _crit_from_u(u, k):
    def bit(i, t):
        cand = t | (jnp.uint32(1) << jnp.uint32(31 - i))
        cnt = jnp.sum((u >= cand).astype(jnp.int32))
        return jnp.where(cnt >= k, cand, t)
    t = lax.fori_loop(0, 32, bit, jnp.uint32(0))

    cnt_gt = jnp.sum((u > t).astype(jnp.int32))
    r = k - cnt_gt
    tie = u == t
    idx = (lax.broadcasted_iota(jnp.int32, (392, 128), 0) * 128
           + lax.broadcasted_iota(jnp.int32, (392, 128), 1))

    def jbit(i, m):
        cand = m | (1 << (16 - i))
        f = jnp.sum((tie & (idx < cand)).astype(jnp.int32))
        return jnp.where(f < r, cand, m)
    m = lax.fori_loop(0, 17, jbit, jnp.int32(0))
    n_r = jnp.where(r > 0, m + 1, 0)

    lane = lax.broadcasted_iota(jnp.int32, (1, 128), 1)
    t_i = lax.bitcast_convert_type(t, jnp.int32)
    return jnp.where(lane == 0, t_i, jnp.where(lane == 1, n_r, 0))


def _layer_body(nparts, k, *refs):
    aggs = refs[:nparts]
    bm_ref, xin_ref = refs[nparts:nparts + 2]
    w1s = refs[nparts + 2:2 * nparts + 2]
    (w2_ref, b_ref, pn_ref, m_ref, h_o, y_o, crit_o, ysc) = \
        refs[2 * nparts + 2:]
    pid = pl.program_id(0)

    @pl.when(pid < NBLK)
    def _():
        h = jnp.maximum(_layer_z(aggs, w1s, bm_ref, xin_ref, w2_ref, b_ref),
                        jnp.float32(0))
        h_o[...] = h
        y = jnp.dot(h, pn_ref[...], preferred_element_type=jnp.float32)
        ym = jnp.where(m_ref[...] > 0, y, jnp.float32(-jnp.inf))
        y_o[...] = ym
        ysc[pl.ds(pid * (BN // 128), BN // 128), :] = ym.reshape(
            BN // 128, 128)

    @pl.when(pid == NBLK)
    def _():
        crit_o[...] = _crit_from_u(_u32(ysc[...]), k)


def _tc_layer(aggs, bm, xin, w1s, w2, b, pn, m, k):
    D = xin.shape[1]
    n = len(aggs)
    cl = lambda i: jnp.minimum(i, NBLK - 1)
    agg_specs = [pl.BlockSpec((2, BN, a.shape[2]), lambda i: (0, cl(i), 0))
                 for a in aggs]
    w1_specs = [pl.BlockSpec(w.shape, lambda i: (0, 0)) for w in w1s]
    return pl.pallas_call(
        functools.partial(_layer_body, n, k),
        grid=(NBLK + 1,),
        in_specs=agg_specs + [
            pl.BlockSpec((BN, 1), lambda i: (cl(i), 0)),
            pl.BlockSpec((BN, D), lambda i: (cl(i), 0)),
        ] + w1_specs + [
            pl.BlockSpec((D, 32), lambda i: (0, 0)),
            pl.BlockSpec((1, 32), lambda i: (0, 0)),
            pl.BlockSpec((32, 1), lambda i: (0, 0)),
            pl.BlockSpec((BN, 1), lambda i: (cl(i), 0)),
        ],
        out_specs=[
            pl.BlockSpec((BN, 32), lambda i: (cl(i), 0)),
            pl.BlockSpec((BN, 1), lambda i: (cl(i), 0)),
            pl.BlockSpec((1, 128), lambda i: (0, 0)),
        ],
        out_shape=[
            jax.ShapeDtypeStruct((N_PAD, 32), jnp.float32),
            jax.ShapeDtypeStruct((N_PAD, 1), jnp.float32),
            jax.ShapeDtypeStruct((1, 128), jnp.int32),
        ],
        scratch_shapes=[pltpu.VMEM((392, 128), jnp.float32)],
    )(*aggs, bm, xin, *w1s, w2, b, pn, m)


def _gate_body(y_ref, h_ref, a0_ref, b0_ref, crit_ref, T_o, H_o, bm_o, m_o):
    pid = pl.program_id(0)
    t_u = lax.bitcast_convert_type(crit_ref[0, 0], jnp.uint32)
    n_r = crit_ref[0, 1]
    y = y_ref[...]
    u = _u32(y)
    rows = pid * BN + lax.broadcasted_iota(jnp.int32, (BN, 1), 0)
    sel = (u > t_u) | ((u == t_u) & (rows < n_r))
    mf = sel.astype(jnp.float32)
    g = jnp.tanh(y) * mf
    Hrow = h_ref[...] * g
    H_o[...] = Hrow
    T_o[...] = Hrow * (a0_ref[...] * mf)
    bm_o[...] = b0_ref[...] * mf
    m_o[...] = mf


def _tc_gate(y, h, a0, b0, crit):
    return pl.pallas_call(
        _gate_body,
        grid=(NBLK,),
        in_specs=[
            pl.BlockSpec((BN, 1), lambda i: (i, 0)),
            pl.BlockSpec((BN, 32), lambda i: (i, 0)),
            pl.BlockSpec((BN, 1), lambda i: (i, 0)),
            pl.BlockSpec((BN, 1), lambda i: (i, 0)),
            pl.BlockSpec((1, 128), lambda i: (0, 0)),
        ],
        out_specs=[
            pl.BlockSpec((BN, 32), lambda i: (i, 0)),
            pl.BlockSpec((BN, 32), lambda i: (i, 0)),
            pl.BlockSpec((BN, 1), lambda i: (i, 0)),
            pl.BlockSpec((BN, 1), lambda i: (i, 0)),
        ],
        out_shape=[
            jax.ShapeDtypeStruct((N_PAD, 32), jnp.float32),
            jax.ShapeDtypeStruct((N_PAD, 32), jnp.float32),
            jax.ShapeDtypeStruct((N_PAD, 1), jnp.float32),
            jax.ShapeDtypeStruct((N_PAD, 1), jnp.float32),
        ],
    )(y, h, a0, b0, crit)


def _final_body(nparts, *refs):
    aggs = refs[:nparts]
    bm_ref, H_ref = refs[nparts:nparts + 2]
    w1s = refs[nparts + 2:2 * nparts + 2]
    m_ref, w2_ref, b_ref, wd_ref, bd_ref, out_o, acc = refs[2 * nparts + 2:]
    pid = pl.program_id(0)

    @pl.when(pid == 0)
    def _():
        acc[...] = jnp.zeros((1, 32), jnp.float32)

    z = _layer_z(aggs, w1s, bm_ref, H_ref, w2_ref, b_ref)
    h2 = jnp.maximum(z, jnp.float32(0)) * m_ref[...]
    acc[...] += jnp.sum(h2, axis=0, keepdims=True)

    @pl.when(pid == NBLK - 1)
    def _():
        pooled = acc[...] / jnp.float32(K2)
        logits = (jnp.dot(pooled, wd_ref[...],
                          preferred_element_type=jnp.float32) + bd_ref[...])
        mx = jnp.max(logits, axis=-1, keepdims=True)
        e = jnp.exp(logits - mx)
        out_o[...] = e / jnp.sum(e, axis=-1, keepdims=True)


def _tc_final(aggs, bm, H, w1s, m, w2, b, wd, bd):
    n = len(aggs)
    agg_specs = [pl.BlockSpec((2, BN, a.shape[2]), lambda i: (0, i, 0))
                 for a in aggs]
    w1_specs = [pl.BlockSpec(w.shape, lambda i: (0, 0)) for w in w1s]
    return pl.pallas_call(
        functools.partial(_final_body, n),
        grid=(NBLK,),
        in_specs=agg_specs + [
            pl.BlockSpec((BN, 1), lambda i: (i, 0)),
            pl.BlockSpec((BN, 32), lambda i: (i, 0)),
        ] + w1_specs + [
            pl.BlockSpec((BN, 1), lambda i: (i, 0)),
            pl.BlockSpec((32, 32), lambda i: (0, 0)),
            pl.BlockSpec((1, 32), lambda i: (0, 0)),
            pl.BlockSpec((32, 3), lambda i: (0, 0)),
            pl.BlockSpec((1, 3), lambda i: (0, 0)),
        ],
        out_specs=pl.BlockSpec((1, 3), lambda i: (0, 0)),
        out_shape=jax.ShapeDtypeStruct((1, 3), jnp.float32),
        scratch_shapes=[pltpu.VMEM((1, 32), jnp.float32)],
    )(*aggs, bm, H, *w1s, m, w2, b, wd, bd)


def kernel(x, W1a, W2a, ba, p, W1b, W2b, bb, W1c, W2c, bc, Wd, bd,
           edge_index, i):
    f32 = jnp.float32
    src = edge_index[0]
    dst = edge_index[1]
    pad = jnp.full((E_PAD - E,), TRASH, jnp.int32)
    srcp = jnp.concatenate([src, pad])
    dstp = jnp.concatenate([dst, pad])
    srcr = srcp.reshape(NW, GRP * NGRP, CHUNK)
    dstr = dstp.reshape(NW, GRP * NGRP, CHUNK)
    srcw = srcp.reshape(NW, GRP_W * NGRP_W, CHUNK_W)
    dstw = dstp.reshape(NW, GRP_W * NGRP_W, CHUNK_W)

    x4 = jnp.zeros((N_PAD, 4), f32).at[:N, :3].set(x)
    W1a4 = jnp.zeros((4, 32), f32).at[:3].set(W1a)
    W2a4 = jnp.zeros((4, 32), f32).at[:3].set(W2a)
    pn = (p / jnp.linalg.norm(p)).reshape(32, 1)
    valid = (jnp.arange(N_PAD) < N).astype(f32).reshape(N_PAD, 1)
    zeros1 = jnp.zeros((N_PAD,), f32)
    zeros4 = jnp.zeros((N_PAD, 4), f32)
    zeros32 = jnp.zeros((N_PAD, 32), f32)

    s0, s1, d0, d1 = _sc_deg(srcw, dstw, zeros1)
    a0, b0, T0 = _tc_deg(s0, s1, d0, d1, x4)
    agg0 = _sc_agg(srcw, dstw, T0, zeros4)
    h, y1, crit1 = _tc_layer([agg0], b0, x4, [W1a4], W2a4,
                             ba.reshape(1, 32), pn, valid, K1)
    T1, H1, bm1, m1 = _tc_gate(y1, h, a0, b0, crit1)
    agg1 = [_sc_agg(srcr, dstr, T1, zeros32)]
    h1, y2, crit2 = _tc_layer(agg1, bm1, H1, [W1b], W2b,
                              bb.reshape(1, 32), pn, m1, K2)
    T2, H2, bm2, m2 = _tc_gate(y2, h1, a0, b0, crit2)
    agg2 = [_sc_agg(srcr, dstr, T2, zeros32)]
    return _tc_final(agg2, bm2, H2, [W1c], m2, W2c,
                     bc.reshape(1, 32), Wd, bd.reshape(1, 3))


# speedup vs baseline: 58.4768x; 1.0002x over previous
import functools

import jax
import jax.numpy as jnp
from jax import lax
from jax.experimental import pallas as pl
from jax.experimental.pallas import tpu as pltpu
from jax.experimental.pallas import tpu_sc as plsc

N = 50000
E = 1600000
N_PAD = 50176
NW = 32
CHUNK = 256
GRP = 4
NGRP = 49
CHUNK_W = 1024
GRP_W = 7
NGRP_W = 7
E_PER_TILE = CHUNK * GRP * NGRP
E_PAD = NW * E_PER_TILE
ROWS_PER_TILE = N_PAD // 16
CP = 112
TRASH = N
K1 = 25000
K2 = 12500
BN = 1024
NBLK = N_PAD // BN

@functools.cache
def _mesh():
    return plsc.VectorSubcoreMesh(core_axis_name="c", subcore_axis_name="s")


def _deg_body(srcr, dstr, zeros1, outS0, outS1, outD0, outD1,
              degS, degD, ones_v, idx_v, buf_v, dsem):
    c = lax.axis_index("c")
    s = lax.axis_index("s")
    w = c * 16 + s

    def setones(j, _):
        ones_v[pl.ds(j * 16, 16)] = jnp.ones((16,), jnp.float32)
        return 0
    lax.fori_loop(0, CHUNK_W // 16, setones, 0)

    row0 = s * ROWS_PER_TILE
    pltpu.sync_copy(zeros1.at[pl.ds(0, CP)], buf_v)

    def zchunk(t, _):
        pltpu.sync_copy(buf_v, degS.at[pl.ds(row0 + t * CP, CP)])
        pltpu.sync_copy(buf_v, degD.at[pl.ds(row0 + t * CP, CP)])
        return 0
    lax.fori_loop(0, ROWS_PER_TILE // CP, zchunk, 0)
    plsc.subcore_barrier()

    def grp(g, _):
        pltpu.sync_copy(srcr.at[w, pl.ds(g * GRP_W, GRP_W)], idx_v)
        descs = [pltpu.async_copy(ones_v, degS.at[idx_v.at[j]], dsem,
                                  add=True) for j in range(GRP_W)]
        for dsc in descs:
            dsc.wait()
        pltpu.sync_copy(dstr.at[w, pl.ds(g * GRP_W, GRP_W)], idx_v)
        descs = [pltpu.async_copy(ones_v, degD.at[idx_v.at[j]], dsem,
                                  add=True) for j in range(GRP_W)]
        for dsc in descs:
            dsc.wait()
        return 0
    lax.fori_loop(0, NGRP_W, grp, 0)

    plsc.subcore_barrier()

    def out1(acc, dstref):
        def cp(t, _):
            sl = pl.ds(row0 + t * CP, CP)
            pltpu.sync_copy(acc.at[sl], buf_v)
            pltpu.sync_copy(buf_v, dstref.at[sl])
            return 0
        lax.fori_loop(0, ROWS_PER_TILE // CP, cp, 0)

    @pl.when(c == 0)
    def _():
        out1(degS, outS0)
        out1(degD, outD0)

    @pl.when(c == 1)
    def _():
        out1(degS, outS1)
        out1(degD, outD1)


@functools.cache
def _sc_deg_fn():
    return pl.kernel(
        _deg_body,
        out_type=[jax.ShapeDtypeStruct((N_PAD,), jnp.float32)] * 4,
        mesh=_mesh(),
        compiler_params=pltpu.CompilerParams(use_tc_tiling_on_sc=False),
        scratch_types=[
            pltpu.VMEM_SHARED((N_PAD,), jnp.float32),
            pltpu.VMEM_SHARED((N_PAD,), jnp.float32),
            pltpu.VMEM((CHUNK_W,), jnp.float32),
            pltpu.VMEM((GRP_W, CHUNK_W), jnp.int32),
            pltpu.VMEM((CP,), jnp.float32),
            pltpu.SemaphoreType.DMA,
        ],
    )


def _sc_deg(srcr, dstr, zeros1):
    return _sc_deg_fn()(srcr, dstr, zeros1)


def _agg_body(D, chk, grpn, ngrp, srcr, dstr, tab, zerosD, out, acc, idxs_v,
              idxd_v, r0_v, r1_v, buf_v, gs0, gs1, ss0, ss1):
    c = lax.axis_index("c")
    s = lax.axis_index("s")
    w = c * 16 + s

    row0 = s * ROWS_PER_TILE
    pltpu.sync_copy(zerosD.at[pl.ds(0, CP)], buf_v)

    def zchunk(t, _):
        pltpu.sync_copy(buf_v, acc.at[pl.ds(row0 + t * CP, CP)])
        return 0
    lax.fori_loop(0, ROWS_PER_TILE // CP, zchunk, 0)
    plsc.subcore_barrier()

    rows = (r0_v, r1_v)
    gsem = (gs0, gs1)
    ssem = (ss0, ss1)

    nsub = CHUNK_W // chk

    def isl(j):
        return (j // nsub, pl.ds((j % nsub) * chk, chk))

    def grp(g, _):
        pltpu.sync_copy(srcr.at[w, pl.ds(g * grpn, grpn)], idxs_v)
        pltpu.sync_copy(dstr.at[w, pl.ds(g * grpn, grpn)], idxd_v)
        nch = grpn * nsub
        gd = [None, None]
        sd = [None, None]
        gd[0] = pltpu.async_copy(tab.at[idxs_v.at[isl(0)]], rows[0], gsem[0])
        for j in range(nch):
            b = j % 2
            gd[b].wait()
            if j >= 1:
                sd[1 - b].wait()
            sd[b] = pltpu.async_copy(rows[b], acc.at[idxd_v.at[isl(j)]],
                                     ssem[b], add=True)
            if j + 1 < nch:
                gd[1 - b] = pltpu.async_copy(tab.at[idxs_v.at[isl(j + 1)]],
                                             rows[1 - b], gsem[1 - b])
        sd[(nch - 1) % 2].wait()
        return 0
    lax.fori_loop(0, ngrp, grp, 0)

    plsc.subcore_barrier()

    def cp(t, _):
        sl = pl.ds(row0 + t * CP, CP)
        pltpu.sync_copy(acc.at[sl], buf_v)
        pltpu.sync_copy(buf_v, out.at[c, sl])
        return 0
    lax.fori_loop(0, ROWS_PER_TILE // CP, cp, 0)


@functools.cache
def _sc_agg_fn(D, chk, grpn, ngrp):
    return pl.kernel(
        functools.partial(_agg_body, D, chk, grpn, ngrp),
        out_type=jax.ShapeDtypeStruct((2, N_PAD, D), jnp.float32),
        mesh=_mesh(),
        compiler_params=pltpu.CompilerParams(use_tc_tiling_on_sc=False),
        scratch_types=[
            pltpu.VMEM_SHARED((N_PAD, D), jnp.float32),
            pltpu.VMEM((grpn, CHUNK_W), jnp.int32),
            pltpu.VMEM((grpn, CHUNK_W), jnp.int32),
            pltpu.VMEM((chk, D), jnp.float32),
            pltpu.VMEM((chk, D), jnp.float32),
            pltpu.VMEM((CP, D), jnp.float32),
            pltpu.SemaphoreType.DMA,
            pltpu.SemaphoreType.DMA,
            pltpu.SemaphoreType.DMA,
            pltpu.SemaphoreType.DMA,
        ],
    )


def _sc_agg(srcr, dstr, tab, zerosD, chk):
    grpn = 1 if chk == CHUNK else GRP_W
    ngrp = NGRP_W * GRP_W // grpn
    return _sc_agg_fn(tab.shape[1], chk, grpn, ngrp)(srcr, dstr, tab, zerosD)


def _u32(y):
    u = lax.bitcast_convert_type(y, jnp.uint32)
    return jnp.where((u >> jnp.uint32(31)) > jnp.uint32(0),
                     ~u, u | jnp.uint32(0x80000000))


def _deg_tc_body(s0_ref, s1_ref, d0_ref, d1_ref, x4_ref, a0_o, b0_o, t0_o):
    ds = s0_ref[...] + s1_ref[...]
    dd = d0_ref[...] + d1_ref[...]
    a0 = jnp.where(ds > 0, lax.rsqrt(jnp.maximum(ds, 1e-12)),
                   jnp.float32(0))
    b0 = jnp.where(dd > 0, lax.rsqrt(jnp.maximum(dd, 1e-12)),
                   jnp.float32(0))
    a0_o[...] = a0
    b0_o[...] = b0
    t0_o[...] = x4_ref[...] * a0


def _tc_deg(s0, s1, d0, d1, x4):
    return pl.pallas_call(
        _deg_tc_body,
        grid=(NBLK,),
        in_specs=[
            pl.BlockSpec((BN, 1), lambda i: (i, 0)),
            pl.BlockSpec((BN, 1), lambda i: (i, 0)),
            pl.BlockSpec((BN, 1), lambda i: (i, 0)),
            pl.BlockSpec((BN, 1), lambda i: (i, 0)),
            pl.BlockSpec((BN, 4), lambda i: (i, 0)),
        ],
        out_specs=[
            pl.BlockSpec((BN, 1), lambda i: (i, 0)),
            pl.BlockSpec((BN, 1), lambda i: (i, 0)),
            pl.BlockSpec((BN, 4), lambda i: (i, 0)),
        ],
        out_shape=[
            jax.ShapeDtypeStruct((N_PAD, 1), jnp.float32),
            jax.ShapeDtypeStruct((N_PAD, 1), jnp.float32),
            jax.ShapeDtypeStruct((N_PAD, 4), jnp.float32),
        ],
    )(s0.reshape(N_PAD, 1), s1.reshape(N_PAD, 1),
      d0.reshape(N_PAD, 1), d1.reshape(N_PAD, 1), x4)


def _layer_z(aggs, w1s, bm_ref, xin_ref, w2_ref, b_ref):
    z = (jnp.dot(xin_ref[...], w2_ref[...],
                 preferred_element_type=jnp.float32) + b_ref[...])
    for ap, w1 in zip(aggs, w1s):
        agg = ap[0] + ap[1]
        z = z + jnp.dot(agg * bm_ref[...], w1[...],
                        preferred_element_type=jnp.float32)
    return z


def _crit_from_u(u, k):
    def bit(i, t):
        cand = t | (jnp.uint32(1) << jnp.uint32(31 - i))
        cnt = jnp.sum((u >= cand).astype(jnp.int32))
        return jnp.where(cnt >= k, cand, t)
    t = lax.fori_loop(0, 32, bit, jnp.uint32(0))

    cnt_gt = jnp.sum((u > t).astype(jnp.int32))
    r = k - cnt_gt
    tie = u == t
    idx = (lax.broadcasted_iota(jnp.int32, (392, 128), 0) * 128
           + lax.broadcasted_iota(jnp.int32, (392, 128), 1))

    def jbit(i, m):
        cand = m | (1 << (16 - i))
        f = jnp.sum((tie & (idx < cand)).astype(jnp.int32))
        return jnp.where(f < r, cand, m)
    m = lax.fori_loop(0, 17, jbit, jnp.int32(0))
    n_r = jnp.where(r > 0, m + 1, 0)

    lane = lax.broadcasted_iota(jnp.int32, (1, 128), 1)
    t_i = lax.bitcast_convert_type(t, jnp.int32)
    return jnp.where(lane == 0, t_i, jnp.where(lane == 1, n_r, 0))


def _layer_body(nparts, k, *refs):
    aggs = refs[:nparts]
    bm_ref, xin_ref = refs[nparts:nparts + 2]
    w1s = refs[nparts + 2:2 * nparts + 2]
    (w2_ref, b_ref, pn_ref, m_ref, h_o, y_o, crit_o, ysc) = \
        refs[2 * nparts + 2:]
    pid = pl.program_id(0)

    @pl.when(pid < NBLK)
    def _():
        h = jnp.maximum(_layer_z(aggs, w1s, bm_ref, xin_ref, w2_ref, b_ref),
                        jnp.float32(0))
        h_o[...] = h
        y = jnp.dot(h, pn_ref[...], preferred_element_type=jnp.float32)
        ym = jnp.where(m_ref[...] > 0, y, jnp.float32(-jnp.inf))
        y_o[...] = ym
        ysc[pl.ds(pid * (BN // 128), BN // 128), :] = ym.reshape(
            BN // 128, 128)

    @pl.when(pid == NBLK)
    def _():
        crit_o[...] = _crit_from_u(_u32(ysc[...]), k)


def _tc_layer(aggs, bm, xin, w1s, w2, b, pn, m, k):
    D = xin.shape[1]
    n = len(aggs)
    cl = lambda i: jnp.minimum(i, NBLK - 1)
    agg_specs = [pl.BlockSpec((2, BN, a.shape[2]), lambda i: (0, cl(i), 0))
                 for a in aggs]
    w1_specs = [pl.BlockSpec(w.shape, lambda i: (0, 0)) for w in w1s]
    return pl.pallas_call(
        functools.partial(_layer_body, n, k),
        grid=(NBLK + 1,),
        in_specs=agg_specs + [
            pl.BlockSpec((BN, 1), lambda i: (cl(i), 0)),
            pl.BlockSpec((BN, D), lambda i: (cl(i), 0)),
        ] + w1_specs + [
            pl.BlockSpec((D, 32), lambda i: (0, 0)),
            pl.BlockSpec((1, 32), lambda i: (0, 0)),
            pl.BlockSpec((32, 1), lambda i: (0, 0)),
            pl.BlockSpec((BN, 1), lambda i: (cl(i), 0)),
        ],
        out_specs=[
            pl.BlockSpec((BN, 32), lambda i: (cl(i), 0)),
            pl.BlockSpec((BN, 1), lambda i: (cl(i), 0)),
            pl.BlockSpec((1, 128), lambda i: (0, 0)),
        ],
        out_shape=[
            jax.ShapeDtypeStruct((N_PAD, 32), jnp.float32),
            jax.ShapeDtypeStruct((N_PAD, 1), jnp.float32),
            jax.ShapeDtypeStruct((1, 128), jnp.int32),
        ],
        scratch_shapes=[pltpu.VMEM((392, 128), jnp.float32)],
    )(*aggs, bm, xin, *w1s, w2, b, pn, m)


def _gate_body(y_ref, h_ref, a0_ref, b0_ref, crit_ref, T_o, H_o, bm_o, m_o):
    pid = pl.program_id(0)
    t_u = lax.bitcast_convert_type(crit_ref[0, 0], jnp.uint32)
    n_r = crit_ref[0, 1]
    y = y_ref[...]
    u = _u32(y)
    rows = pid * BN + lax.broadcasted_iota(jnp.int32, (BN, 1), 0)
    sel = (u > t_u) | ((u == t_u) & (rows < n_r))
    mf = sel.astype(jnp.float32)
    g = jnp.tanh(y) * mf
    Hrow = h_ref[...] * g
    H_o[...] = Hrow
    T_o[...] = Hrow * (a0_ref[...] * mf)
    bm_o[...] = b0_ref[...] * mf
    m_o[...] = mf


def _tc_gate(y, h, a0, b0, crit):
    return pl.pallas_call(
        _gate_body,
        grid=(NBLK,),
        in_specs=[
            pl.BlockSpec((BN, 1), lambda i: (i, 0)),
            pl.BlockSpec((BN, 32), lambda i: (i, 0)),
            pl.BlockSpec((BN, 1), lambda i: (i, 0)),
            pl.BlockSpec((BN, 1), lambda i: (i, 0)),
            pl.BlockSpec((1, 128), lambda i: (0, 0)),
        ],
        out_specs=[
            pl.BlockSpec((BN, 32), lambda i: (i, 0)),
            pl.BlockSpec((BN, 32), lambda i: (i, 0)),
            pl.BlockSpec((BN, 1), lambda i: (i, 0)),
            pl.BlockSpec((BN, 1), lambda i: (i, 0)),
        ],
        out_shape=[
            jax.ShapeDtypeStruct((N_PAD, 32), jnp.float32),
            jax.ShapeDtypeStruct((N_PAD, 32), jnp.float32),
            jax.ShapeDtypeStruct((N_PAD, 1), jnp.float32),
            jax.ShapeDtypeStruct((N_PAD, 1), jnp.float32),
        ],
    )(y, h, a0, b0, crit)


def _final_body(nparts, *refs):
    aggs = refs[:nparts]
    bm_ref, H_ref = refs[nparts:nparts + 2]
    w1s = refs[nparts + 2:2 * nparts + 2]
    m_ref, w2_ref, b_ref, wd_ref, bd_ref, out_o, acc = refs[2 * nparts + 2:]
    pid = pl.program_id(0)

    @pl.when(pid == 0)
    def _():
        acc[...] = jnp.zeros((1, 32), jnp.float32)

    z = _layer_z(aggs, w1s, bm_ref, H_ref, w2_ref, b_ref)
    h2 = jnp.maximum(z, jnp.float32(0)) * m_ref[...]
    acc[...] += jnp.sum(h2, axis=0, keepdims=True)

    @pl.when(pid == NBLK - 1)
    def _():
        pooled = acc[...] / jnp.float32(K2)
        logits = (jnp.dot(pooled, wd_ref[...],
                          preferred_element_type=jnp.float32) + bd_ref[...])
        mx = jnp.max(logits, axis=-1, keepdims=True)
        e = jnp.exp(logits - mx)
        out_o[...] = e / jnp.sum(e, axis=-1, keepdims=True)


def _tc_final(aggs, bm, H, w1s, m, w2, b, wd, bd):
    n = len(aggs)
    agg_specs = [pl.BlockSpec((2, BN, a.shape[2]), lambda i: (0, i, 0))
                 for a in aggs]
    w1_specs = [pl.BlockSpec(w.shape, lambda i: (0, 0)) for w in w1s]
    return pl.pallas_call(
        functools.partial(_final_body, n),
        grid=(NBLK,),
        in_specs=agg_specs + [
            pl.BlockSpec((BN, 1), lambda i: (i, 0)),
            pl.BlockSpec((BN, 32), lambda i: (i, 0)),
        ] + w1_specs + [
            pl.BlockSpec((BN, 1), lambda i: (i, 0)),
            pl.BlockSpec((32, 32), lambda i: (0, 0)),
            pl.BlockSpec((1, 32), lambda i: (0, 0)),
            pl.BlockSpec((32, 3), lambda i: (0, 0)),
            pl.BlockSpec((1, 3), lambda i: (0, 0)),
        ],
        out_specs=pl.BlockSpec((1, 3), lambda i: (0, 0)),
        out_shape=jax.ShapeDtypeStruct((1, 3), jnp.float32),
        scratch_shapes=[pltpu.VMEM((1, 32), jnp.float32)],
    )(*aggs, bm, H, *w1s, m, w2, b, wd, bd)


def kernel(x, W1a, W2a, ba, p, W1b, W2b, bb, W1c, W2c, bc, Wd, bd,
           edge_index, i):
    f32 = jnp.float32
    src = edge_index[0]
    dst = edge_index[1]
    pad = jnp.full((E_PAD - E,), TRASH, jnp.int32)
    srcw = jnp.concatenate([src, pad]).reshape(NW, GRP_W * NGRP_W, CHUNK_W)
    dstw = jnp.concatenate([dst, pad]).reshape(NW, GRP_W * NGRP_W, CHUNK_W)

    x4 = jnp.zeros((N_PAD, 4), f32).at[:N, :3].set(x)
    W1a4 = jnp.zeros((4, 32), f32).at[:3].set(W1a)
    W2a4 = jnp.zeros((4, 32), f32).at[:3].set(W2a)
    pn = (p / jnp.linalg.norm(p)).reshape(32, 1)
    valid = (jnp.arange(N_PAD) < N).astype(f32).reshape(N_PAD, 1)
    zeros1 = jnp.zeros((N_PAD,), f32)
    zeros4 = jnp.zeros((N_PAD, 4), f32)
    zeros32 = jnp.zeros((N_PAD, 32), f32)

    s0, s1, d0, d1 = _sc_deg(srcw, dstw, zeros1)
    a0, b0, T0 = _tc_deg(s0, s1, d0, d1, x4)
    agg0 = _sc_agg(srcw, dstw, T0, zeros4, CHUNK_W)
    h, y1, crit1 = _tc_layer([agg0], b0, x4, [W1a4], W2a4,
                             ba.reshape(1, 32), pn, valid, K1)
    T1, H1, bm1, m1 = _tc_gate(y1, h, a0, b0, crit1)
    agg1 = [_sc_agg(srcw, dstw, T1, zeros32, CHUNK)]
    h1, y2, crit2 = _tc_layer(agg1, bm1, H1, [W1b], W2b,
                              bb.reshape(1, 32), pn, m1, K2)
    T2, H2, bm2, m2 = _tc_gate(y2, h1, a0, b0, crit2)
    agg2 = [_sc_agg(srcw, dstw, T2, zeros32, CHUNK)]
    return _tc_final(agg2, bm2, H2, [W1c], m2, W2c,
                     bc.reshape(1, 32), Wd, bd.reshape(1, 3))
